# baseline (speedup 1.0000x reference)
import functools

import jax
import jax.numpy as jnp
from jax import lax
from jax.experimental import pallas as pl
from jax.experimental.pallas import tpu as pltpu
from jax.experimental.pallas import tpu_sc as plsc

N = 6890
E = 41340
KS = 5
KC = 125
NPAD = 6912
B = 256
NP = 8 * E
CHUNK = 8192
P_CAP = 368640
NBLK = P_CAP // B
NC, NS, LANES = 2, 16, 16
NW = NC * NS


def _prepare_pairs(pseudo, edge_index):
    src = edge_index[0]
    dst = edge_index[1]
    u = pseudo * (KS - 1)
    lo_f = jnp.clip(jnp.floor(u), 0.0, KS - 2)
    frac = u - lo_f
    lo = lo_f.astype(jnp.int32)

    ws, idxs = [], []
    for bits in range(8):
        w = jnp.ones((E,), jnp.float32)
        idx = jnp.zeros((E,), jnp.int32)
        for d in range(3):
            sel = (bits >> d) & 1
            w = w * (frac[:, d] if sel else 1.0 - frac[:, d])
            idx = idx * KS + (lo[:, d] + sel)
        ws.append(w)
        idxs.append(idx)
    w8 = jnp.concatenate(ws)
    k8 = jnp.concatenate(idxs)
    s8 = jnp.tile(src, 8)
    d8 = jnp.tile(dst, 8)

    counts = jnp.zeros((KC,), jnp.int32).at[k8].add(1)
    padded = ((counts + (B - 1)) // B) * B
    offs = jnp.concatenate([jnp.zeros((1,), jnp.int32),
                            jnp.cumsum(padded)[:-1].astype(jnp.int32)])
    cum_counts = jnp.concatenate([jnp.zeros((1,), jnp.int32),
                                  jnp.cumsum(counts)[:-1].astype(jnp.int32)])

    order = jnp.argsort(k8)
    k_sorted = k8[order]
    rank = jnp.arange(NP, dtype=jnp.int32) - cum_counts[k_sorted]
    pos = offs[k_sorted] + rank

    fill = (jnp.arange(P_CAP, dtype=jnp.int32) % N)
    s_sorted = fill.at[pos].set(s8[order])
    d_sorted = fill.at[pos].set(d8[order])
    w_sorted = jnp.zeros((P_CAP,), jnp.float32).at[pos].set(w8[order])

    blk_start = jnp.arange(NBLK, dtype=jnp.int32) * B
    block_k = jnp.sum(offs[None, :] <= blk_start[:, None], axis=1).astype(jnp.int32) - 1

    deg = jnp.zeros((N,), jnp.float32).at[dst].add(1.0)
    deg = jnp.maximum(deg, 1.0)
    deg = jnp.concatenate([deg, jnp.ones((NPAD - N,), jnp.float32)])
    return s_sorted, d_sorted, w_sorted, block_k, deg.reshape(1, NPAD)


def _sc_gather(din):
    mesh = plsc.VectorSubcoreMesh(core_axis_name="c", subcore_axis_name="s",
                                  num_cores=NC, num_subcores=NS)
    n_chunks = P_CAP // CHUNK

    if din >= NW:
        R = din // NW

        @functools.partial(
            pl.kernel, mesh=mesh,
            compiler_params=pltpu.CompilerParams(needs_layout_passes=False),
            out_type=jax.ShapeDtypeStruct((din, P_CAP), jnp.float32),
            scratch_types=[
                pltpu.VMEM((R * NPAD,), jnp.float32),
                pltpu.VMEM((CHUNK,), jnp.int32),
                pltpu.VMEM((R, CHUNK), jnp.float32),
            ],
        )
        def k(xt_hbm, s_hbm, xgt_hbm, xrow_v, sidx_v, out_v):
            wid = lax.axis_index("s") * NC + lax.axis_index("c")
            row0 = wid * R
            for r in range(R):
                pltpu.sync_copy(xt_hbm.at[row0 + r],
                                xrow_v.at[pl.ds(r * NPAD, NPAD)])

            def chunk_body(g, _):
                base = g * CHUNK
                pltpu.sync_copy(s_hbm.at[pl.ds(base, CHUNK)], sidx_v)

                def body(t, _):
                    idx = sidx_v[pl.ds(t * LANES, LANES)]
                    for r in range(R):
                        vals = plsc.load_gather(
                            xrow_v, [idx + jnp.int32(r * NPAD)])
                        out_v[r, pl.ds(t * LANES, LANES)] = vals
                    return 0
                lax.fori_loop(0, CHUNK // LANES, body, 0)
                pltpu.sync_copy(out_v, xgt_hbm.at[pl.ds(row0, R), pl.ds(base, CHUNK)])
                return 0
            lax.fori_loop(0, n_chunks, chunk_body, 0)
        return k

    per_w = P_CAP // NW

    @functools.partial(
        pl.kernel, mesh=mesh,
        compiler_params=pltpu.CompilerParams(needs_layout_passes=False),
        out_type=jax.ShapeDtypeStruct((din, P_CAP), jnp.float32),
        scratch_types=[
            pltpu.VMEM((NPAD,), jnp.float32),
            pltpu.VMEM((per_w,), jnp.int32),
            pltpu.VMEM((per_w,), jnp.float32),
        ],
    )
    def k1(xt_hbm, s_hbm, xgt_hbm, xrow_v, sidx_v, out_v):
        wid = lax.axis_index("s") * NC + lax.axis_index("c")
        base = wid * per_w
        pltpu.sync_copy(xt_hbm.at[0], xrow_v)
        pltpu.sync_copy(s_hbm.at[pl.ds(base, per_w)], sidx_v)

        def body(t, _):
            idx = sidx_v[pl.ds(t * LANES, LANES)]
            vals = plsc.load_gather(xrow_v, [idx])
            out_v[pl.ds(t * LANES, LANES)] = vals
            return 0
        lax.fori_loop(0, per_w // LANES, body, 0)
        pltpu.sync_copy(out_v, xgt_hbm.at[0, pl.ds(base, per_w)])
    return k1


def _sc_scatter(dout):
    mesh = plsc.VectorSubcoreMesh(core_axis_name="c", subcore_axis_name="s",
                                  num_cores=NC, num_subcores=NS)
    n_chunks = P_CAP // CHUNK
    R = dout // NW
    assert R >= 1

    @functools.partial(
        pl.kernel, mesh=mesh,
        compiler_params=pltpu.CompilerParams(needs_layout_passes=False),
        out_type=jax.ShapeDtypeStruct((dout, NPAD), jnp.float32),
        scratch_types=[
            pltpu.VMEM((R * NPAD,), jnp.float32),
            pltpu.VMEM((CHUNK,), jnp.int32),
            pltpu.VMEM((R, CHUNK), jnp.float32),
        ],
    )
    def k(mt_hbm, d_hbm, numt_hbm, acc_v, didx_v, m_v):
        wid = lax.axis_index("s") * NC + lax.axis_index("c")
        row0 = wid * R

        def zero_body(t, _):
            acc_v[pl.ds(t * LANES, LANES)] = jnp.zeros((LANES,), jnp.float32)
            return 0
        lax.fori_loop(0, (R * NPAD) // LANES, zero_body, 0)

        def chunk_body(g, _):
            base = g * CHUNK
            pltpu.sync_copy(d_hbm.at[pl.ds(base, CHUNK)], didx_v)
            pltpu.sync_copy(mt_hbm.at[pl.ds(row0, R), pl.ds(base, CHUNK)], m_v)

            def body(t, _):
                idx = didx_v[pl.ds(t * LANES, LANES)]
                for r in range(R):
                    vals = m_v[r, pl.ds(t * LANES, LANES)]
                    plsc.addupdate_scatter(acc_v, [idx + jnp.int32(r * NPAD)], vals)
                return 0
            lax.fori_loop(0, CHUNK // LANES, body, 0)
            return 0
        lax.fori_loop(0, n_chunks, chunk_body, 0)
        for r in range(R):
            pltpu.sync_copy(acc_v.at[pl.ds(r * NPAD, NPAD)], numt_hbm.at[row0 + r])
    return k


def _tc_matmul(din, dout):
    def body(block_k_ref, xg_ref, w_ref, wgt_ref, out_ref):
        wk = w_ref[...]
        xg = xg_ref[...]
        m = lax.dot_general(wk, xg, (((0,), (0,)), ((), ())),
                            preferred_element_type=jnp.float32)
        out_ref[...] = m * wgt_ref[...]

    grid_spec = pltpu.PrefetchScalarGridSpec(
        num_scalar_prefetch=1,
        grid=(NBLK,),
        in_specs=[
            pl.BlockSpec((din, B), lambda j, bk: (0, j)),
            pl.BlockSpec((None, din, dout), lambda j, bk: (bk[j], 0, 0)),
            pl.BlockSpec((None, 1, B), lambda j, bk: (j, 0, 0)),
        ],
        out_specs=pl.BlockSpec((dout, B), lambda j, bk: (0, j)),
    )
    return pl.pallas_call(
        body, grid_spec=grid_spec,
        out_shape=jax.ShapeDtypeStruct((dout, P_CAP), jnp.float32),
    )


TN = 768


def _tc_epilogue(din, dout):
    def body(num_ref, deg_ref, xt_ref, wr_ref, b_ref, out_ref):
        agg = num_ref[...] / deg_ref[...]
        root = lax.dot_general(wr_ref[...], xt_ref[...], (((0,), (0,)), ((), ())),
                               preferred_element_type=jnp.float32)
        h = agg + root + b_ref[...]
        out_ref[...] = jnp.where(h > 0, h, jnp.exp(h) - 1.0)

    return pl.pallas_call(
        body,
        grid=(NPAD // TN,),
        in_specs=[
            pl.BlockSpec((dout, TN), lambda j: (0, j)),
            pl.BlockSpec((1, TN), lambda j: (0, j)),
            pl.BlockSpec((din, TN), lambda j: (0, j)),
            pl.BlockSpec((din, dout), lambda j: (0, 0)),
            pl.BlockSpec((dout, 1), lambda j: (0, 0)),
        ],
        out_specs=pl.BlockSpec((dout, TN), lambda j: (0, j)),
        out_shape=jax.ShapeDtypeStruct((dout, NPAD), jnp.float32),
    )


def _tc_lin1():
    def body(xt_ref, w_ref, b_ref, out_ref):
        h = lax.dot_general(w_ref[...], xt_ref[...], (((0,), (0,)), ((), ())),
                            preferred_element_type=jnp.float32) + b_ref[...]
        out_ref[...] = jnp.where(h > 0, h, jnp.exp(h) - 1.0)

    return pl.pallas_call(
        body,
        grid=(NPAD // TN,),
        in_specs=[
            pl.BlockSpec((64, TN), lambda j: (0, j)),
            pl.BlockSpec((64, 256), lambda j: (0, 0)),
            pl.BlockSpec((256, 1), lambda j: (0, 0)),
        ],
        out_specs=pl.BlockSpec((256, TN), lambda j: (0, j)),
        out_shape=jax.ShapeDtypeStruct((256, NPAD), jnp.float32),
    )


TR = 256


def _tc_head():
    def body(h_ref, w_ref, b_ref, out_ref):
        logits = lax.dot_general(h_ref[...], w_ref[...], (((0,), (0,)), ((), ())),
                                 preferred_element_type=jnp.float32)
        logits = logits + b_ref[...]
        m = jnp.max(logits, axis=1, keepdims=True)
        lse = m + jnp.log(jnp.sum(jnp.exp(logits - m), axis=1, keepdims=True))
        out_ref[...] = logits - lse

    nblk = (N + TR - 1) // TR
    return pl.pallas_call(
        body,
        grid=(nblk,),
        in_specs=[
            pl.BlockSpec((256, TR), lambda j: (0, j)),
            pl.BlockSpec((256, N), lambda j: (0, 0)),
            pl.BlockSpec((1, N), lambda j: (0, 0)),
        ],
        out_specs=pl.BlockSpec((TR, N), lambda j: (j, 0)),
        out_shape=jax.ShapeDtypeStruct((N, N), jnp.float32),
    )


def kernel(x, pseudo, edge_index, W1, Wr1, b1, W2, Wr2, b2, W3, Wr3, b3,
           W4, Wr4, b4, W5, Wr5, b5, W6, Wr6, b6, lin1_W, lin1_b, lin2_W, lin2_b):
    s_sorted, d_sorted, w_sorted, block_k, deg = _prepare_pairs(pseudo, edge_index)
    w_blk = w_sorted.reshape(NBLK, 1, B)

    xt = jnp.zeros((1, NPAD), jnp.float32).at[:, :N].set(x.T)
    convs = [(W1, Wr1, b1), (W2, Wr2, b2), (W3, Wr3, b3),
             (W4, Wr4, b4), (W5, Wr5, b5), (W6, Wr6, b6)]
    for (W, Wr, b) in convs:
        din, dout = W.shape[1], W.shape[2]
        xgt = _sc_gather(din)(xt, s_sorted)
        mt = _tc_matmul(din, dout)(block_k, xgt, W, w_blk)
        numt = _sc_scatter(dout)(mt, d_sorted)
        xt = _tc_epilogue(din, dout)(numt, deg, xt, Wr, b.reshape(dout, 1))

    h1t = _tc_lin1()(xt, lin1_W, lin1_b.reshape(256, 1))
    out = _tc_head()(h1t, lin2_W, lin2_b.reshape(1, N))
    return out


# speedup vs baseline: 1.0887x; 1.0887x over previous
import functools

import jax
import jax.numpy as jnp
from jax import lax
from jax.experimental import pallas as pl
from jax.experimental.pallas import tpu as pltpu
from jax.experimental.pallas import tpu_sc as plsc

N = 6890
E = 41340
KS = 5
KC = 125
NPAD = 6912
B = 256
NP = 8 * E
CHUNK = 8192
P_CAP = 368640
NBLK = P_CAP // B
NC, NS, LANES = 2, 16, 16
NW = NC * NS


def _prepare_pairs(pseudo, edge_index):
    src = edge_index[0]
    dst = edge_index[1]
    u = pseudo * (KS - 1)
    lo_f = jnp.clip(jnp.floor(u), 0.0, KS - 2)
    frac = u - lo_f
    lo = lo_f.astype(jnp.int32)

    ws, idxs = [], []
    for bits in range(8):
        w = jnp.ones((E,), jnp.float32)
        idx = jnp.zeros((E,), jnp.int32)
        for d in range(3):
            sel = (bits >> d) & 1
            w = w * (frac[:, d] if sel else 1.0 - frac[:, d])
            idx = idx * KS + (lo[:, d] + sel)
        ws.append(w)
        idxs.append(idx)
    w8 = jnp.concatenate(ws)
    k8 = jnp.concatenate(idxs)
    s8 = jnp.tile(src, 8)
    d8 = jnp.tile(dst, 8)

    counts = jnp.zeros((KC,), jnp.int32).at[k8].add(1)
    padded = ((counts + (B - 1)) // B) * B
    offs = jnp.concatenate([jnp.zeros((1,), jnp.int32),
                            jnp.cumsum(padded)[:-1].astype(jnp.int32)])
    cum_counts = jnp.concatenate([jnp.zeros((1,), jnp.int32),
                                  jnp.cumsum(counts)[:-1].astype(jnp.int32)])

    order = jnp.argsort(k8)
    k_sorted = k8[order]
    rank = jnp.arange(NP, dtype=jnp.int32) - cum_counts[k_sorted]
    pos = offs[k_sorted] + rank

    fill = (jnp.arange(P_CAP, dtype=jnp.int32) % N)
    s_sorted = fill.at[pos].set(s8[order])
    d_sorted = fill.at[pos].set(d8[order])
    w_sorted = jnp.zeros((P_CAP,), jnp.float32).at[pos].set(w8[order])

    blk_start = jnp.arange(NBLK, dtype=jnp.int32) * B
    block_k = jnp.sum(offs[None, :] <= blk_start[:, None], axis=1).astype(jnp.int32) - 1

    deg = jnp.zeros((N,), jnp.float32).at[dst].add(1.0)
    deg = jnp.maximum(deg, 1.0)
    deg = jnp.concatenate([deg, jnp.ones((NPAD - N,), jnp.float32)])
    return s_sorted, d_sorted, w_sorted, block_k, deg.reshape(1, NPAD)


def _sc_gather(din):
    mesh = plsc.VectorSubcoreMesh(core_axis_name="c", subcore_axis_name="s",
                                  num_cores=NC, num_subcores=NS)
    n_chunks = P_CAP // CHUNK

    if din >= NW:
        R = din // NW

        @functools.partial(
            pl.kernel, mesh=mesh,
            compiler_params=pltpu.CompilerParams(needs_layout_passes=False),
            out_type=jax.ShapeDtypeStruct((din, P_CAP), jnp.float32),
            scratch_types=[
                pltpu.VMEM((R * NPAD,), jnp.float32),
                pltpu.VMEM((CHUNK,), jnp.int32),
                pltpu.VMEM((R, CHUNK), jnp.float32),
            ],
        )
        def k(xt_hbm, s_hbm, xgt_hbm, xrow_v, sidx_v, out_v):
            wid = lax.axis_index("s") * NC + lax.axis_index("c")
            row0 = wid * R
            for r in range(R):
                pltpu.sync_copy(xt_hbm.at[row0 + r],
                                xrow_v.at[pl.ds(r * NPAD, NPAD)])

            def chunk_body(g, _):
                base = g * CHUNK
                pltpu.sync_copy(s_hbm.at[pl.ds(base, CHUNK)], sidx_v)

                @plsc.parallel_loop(0, CHUNK, LANES, unroll=8)
                def body(t):
                    idx = sidx_v[pl.ds(t, LANES)]
                    for r in range(R):
                        vals = plsc.load_gather(
                            xrow_v, [idx + jnp.int32(r * NPAD)])
                        out_v[r, pl.ds(t, LANES)] = vals
                pltpu.sync_copy(out_v, xgt_hbm.at[pl.ds(row0, R), pl.ds(base, CHUNK)])
                return 0
            lax.fori_loop(0, n_chunks, chunk_body, 0)
        return k

    per_w = P_CAP // NW

    @functools.partial(
        pl.kernel, mesh=mesh,
        compiler_params=pltpu.CompilerParams(needs_layout_passes=False),
        out_type=jax.ShapeDtypeStruct((din, P_CAP), jnp.float32),
        scratch_types=[
            pltpu.VMEM((NPAD,), jnp.float32),
            pltpu.VMEM((per_w,), jnp.int32),
            pltpu.VMEM((per_w,), jnp.float32),
        ],
    )
    def k1(xt_hbm, s_hbm, xgt_hbm, xrow_v, sidx_v, out_v):
        wid = lax.axis_index("s") * NC + lax.axis_index("c")
        base = wid * per_w
        pltpu.sync_copy(xt_hbm.at[0], xrow_v)
        pltpu.sync_copy(s_hbm.at[pl.ds(base, per_w)], sidx_v)

        @plsc.parallel_loop(0, per_w, LANES, unroll=8)
        def body(t):
            idx = sidx_v[pl.ds(t, LANES)]
            vals = plsc.load_gather(xrow_v, [idx])
            out_v[pl.ds(t, LANES)] = vals
        pltpu.sync_copy(out_v, xgt_hbm.at[0, pl.ds(base, per_w)])
    return k1


def _sc_scatter(dout):
    mesh = plsc.VectorSubcoreMesh(core_axis_name="c", subcore_axis_name="s",
                                  num_cores=NC, num_subcores=NS)
    n_chunks = P_CAP // CHUNK
    R = dout // NW
    assert R >= 1

    @functools.partial(
        pl.kernel, mesh=mesh,
        compiler_params=pltpu.CompilerParams(needs_layout_passes=False),
        out_type=jax.ShapeDtypeStruct((dout, NPAD), jnp.float32),
        scratch_types=[
            pltpu.VMEM((R * NPAD,), jnp.float32),
            pltpu.VMEM((CHUNK,), jnp.int32),
            pltpu.VMEM((R, CHUNK), jnp.float32),
        ],
    )
    def k(mt_hbm, d_hbm, numt_hbm, acc_v, didx_v, m_v):
        wid = lax.axis_index("s") * NC + lax.axis_index("c")
        row0 = wid * R

        @plsc.parallel_loop(0, R * NPAD, LANES, unroll=8)
        def zero_body(t):
            acc_v[pl.ds(t, LANES)] = jnp.zeros((LANES,), jnp.float32)

        def chunk_body(g, _):
            base = g * CHUNK
            pltpu.sync_copy(d_hbm.at[pl.ds(base, CHUNK)], didx_v)
            pltpu.sync_copy(mt_hbm.at[pl.ds(row0, R), pl.ds(base, CHUNK)], m_v)

            @plsc.parallel_loop(0, CHUNK, LANES, unroll=8)
            def body(t):
                idx = didx_v[pl.ds(t, LANES)]
                for r in range(R):
                    vals = m_v[r, pl.ds(t, LANES)]
                    plsc.addupdate_scatter(acc_v, [idx + jnp.int32(r * NPAD)], vals)
            return 0
        lax.fori_loop(0, n_chunks, chunk_body, 0)
        for r in range(R):
            pltpu.sync_copy(acc_v.at[pl.ds(r * NPAD, NPAD)], numt_hbm.at[row0 + r])
    return k


def _tc_matmul(din, dout):
    def body(block_k_ref, xg_ref, w_ref, wgt_ref, out_ref):
        wk = w_ref[...]
        xg = xg_ref[...]
        m = lax.dot_general(wk, xg, (((0,), (0,)), ((), ())),
                            preferred_element_type=jnp.float32)
        out_ref[...] = m * wgt_ref[...]

    grid_spec = pltpu.PrefetchScalarGridSpec(
        num_scalar_prefetch=1,
        grid=(NBLK,),
        in_specs=[
            pl.BlockSpec((din, B), lambda j, bk: (0, j)),
            pl.BlockSpec((None, din, dout), lambda j, bk: (bk[j], 0, 0)),
            pl.BlockSpec((None, 1, B), lambda j, bk: (j, 0, 0)),
        ],
        out_specs=pl.BlockSpec((dout, B), lambda j, bk: (0, j)),
    )
    return pl.pallas_call(
        body, grid_spec=grid_spec,
        out_shape=jax.ShapeDtypeStruct((dout, P_CAP), jnp.float32),
    )


TN = 768


def _tc_epilogue(din, dout):
    def body(num_ref, deg_ref, xt_ref, wr_ref, b_ref, out_ref):
        agg = num_ref[...] / deg_ref[...]
        root = lax.dot_general(wr_ref[...], xt_ref[...], (((0,), (0,)), ((), ())),
                               preferred_element_type=jnp.float32)
        h = agg + root + b_ref[...]
        out_ref[...] = jnp.where(h > 0, h, jnp.exp(h) - 1.0)

    return pl.pallas_call(
        body,
        grid=(NPAD // TN,),
        in_specs=[
            pl.BlockSpec((dout, TN), lambda j: (0, j)),
            pl.BlockSpec((1, TN), lambda j: (0, j)),
            pl.BlockSpec((din, TN), lambda j: (0, j)),
            pl.BlockSpec((din, dout), lambda j: (0, 0)),
            pl.BlockSpec((dout, 1), lambda j: (0, 0)),
        ],
        out_specs=pl.BlockSpec((dout, TN), lambda j: (0, j)),
        out_shape=jax.ShapeDtypeStruct((dout, NPAD), jnp.float32),
    )


def _tc_lin1():
    def body(xt_ref, w_ref, b_ref, out_ref):
        h = lax.dot_general(w_ref[...], xt_ref[...], (((0,), (0,)), ((), ())),
                            preferred_element_type=jnp.float32) + b_ref[...]
        out_ref[...] = jnp.where(h > 0, h, jnp.exp(h) - 1.0)

    return pl.pallas_call(
        body,
        grid=(NPAD // TN,),
        in_specs=[
            pl.BlockSpec((64, TN), lambda j: (0, j)),
            pl.BlockSpec((64, 256), lambda j: (0, 0)),
            pl.BlockSpec((256, 1), lambda j: (0, 0)),
        ],
        out_specs=pl.BlockSpec((256, TN), lambda j: (0, j)),
        out_shape=jax.ShapeDtypeStruct((256, NPAD), jnp.float32),
    )


TR = 256


def _tc_head():
    def body(h_ref, w_ref, b_ref, out_ref):
        logits = lax.dot_general(h_ref[...], w_ref[...], (((0,), (0,)), ((), ())),
                                 preferred_element_type=jnp.float32)
        logits = logits + b_ref[...]
        m = jnp.max(logits, axis=1, keepdims=True)
        lse = m + jnp.log(jnp.sum(jnp.exp(logits - m), axis=1, keepdims=True))
        out_ref[...] = logits - lse

    nblk = (N + TR - 1) // TR
    return pl.pallas_call(
        body,
        grid=(nblk,),
        in_specs=[
            pl.BlockSpec((256, TR), lambda j: (0, j)),
            pl.BlockSpec((256, N), lambda j: (0, 0)),
            pl.BlockSpec((1, N), lambda j: (0, 0)),
        ],
        out_specs=pl.BlockSpec((TR, N), lambda j: (j, 0)),
        out_shape=jax.ShapeDtypeStruct((N, N), jnp.float32),
    )


def kernel(x, pseudo, edge_index, W1, Wr1, b1, W2, Wr2, b2, W3, Wr3, b3,
           W4, Wr4, b4, W5, Wr5, b5, W6, Wr6, b6, lin1_W, lin1_b, lin2_W, lin2_b):
    s_sorted, d_sorted, w_sorted, block_k, deg = _prepare_pairs(pseudo, edge_index)
    w_blk = w_sorted.reshape(NBLK, 1, B)

    xt = jnp.zeros((1, NPAD), jnp.float32).at[:, :N].set(x.T)
    convs = [(W1, Wr1, b1), (W2, Wr2, b2), (W3, Wr3, b3),
             (W4, Wr4, b4), (W5, Wr5, b5), (W6, Wr6, b6)]
    for (W, Wr, b) in convs:
        din, dout = W.shape[1], W.shape[2]
        xgt = _sc_gather(din)(xt, s_sorted)
        mt = _tc_matmul(din, dout)(block_k, xgt, W, w_blk)
        numt = _sc_scatter(dout)(mt, d_sorted)
        xt = _tc_epilogue(din, dout)(numt, deg, xt, Wr, b.reshape(dout, 1))

    h1t = _tc_lin1()(xt, lin1_W, lin1_b.reshape(256, 1))
    out = _tc_head()(h1t, lin2_W, lin2_b.reshape(1, N))
    return out


# speedup vs baseline: 3.0258x; 2.7793x over previous
import functools

import jax
import jax.numpy as jnp
from jax import lax
from jax.experimental import pallas as pl
from jax.experimental.pallas import tpu as pltpu
from jax.experimental.pallas import tpu_sc as plsc

N = 6890
E = 41340
KS = 5
KC = 125
NPAD = 6912
B = 256
NP = 8 * E
CHUNK = 8192
P_CAP = 368640
NBLK = P_CAP // B
NC, NS, LANES = 2, 16, 16
NW = NC * NS


E_PAD = 41472
EW = E_PAD // NW
NGRP = EW // LANES
NP_PAD = 8 * E_PAD
PT = NP_PAD // NS
CH2 = 6912
HALF = P_CAP // 2
HALF16 = HALF // NS
NBLK_PAD = 1536
OFFC = [(c & 1) * 25 + ((c >> 1) & 1) * 5 + ((c >> 2) & 1) for c in range(8)]

_SC_MESH = dict(core_axis_name="c", subcore_axis_name="s",
                num_cores=NC, num_subcores=NS)
_SC_PARAMS = dict(compiler_params=pltpu.CompilerParams(needs_layout_passes=False))


def _corner_w(c, f0, f1, f2):
    t0 = f0 if (c & 1) else 1.0 - f0
    t1 = f1 if ((c >> 1) & 1) else 1.0 - f1
    t2 = f2 if ((c >> 2) & 1) else 1.0 - f2
    return t0 * t1 * t2


def _spline_loop_a(ps_v, f0_v, f1_v, f2_v, lb_v, iota):
    @plsc.parallel_loop(0, EW, LANES, unroll=4)
    def la(t):
        idx3 = (t + iota) * 3
        u0 = plsc.load_gather(ps_v, [idx3]) * (KS - 1.0)
        u1 = plsc.load_gather(ps_v, [idx3 + 1]) * (KS - 1.0)
        u2 = plsc.load_gather(ps_v, [idx3 + 2]) * (KS - 1.0)
        l0 = jnp.minimum(u0.astype(jnp.int32), KS - 2)
        l1 = jnp.minimum(u1.astype(jnp.int32), KS - 2)
        l2 = jnp.minimum(u2.astype(jnp.int32), KS - 2)
        f0_v[pl.ds(t, LANES)] = u0 - l0.astype(jnp.float32)
        f1_v[pl.ds(t, LANES)] = u1 - l1.astype(jnp.float32)
        f2_v[pl.ds(t, LANES)] = u2 - l2.astype(jnp.float32)
        lb_v[pl.ds(t, LANES)] = l0 * 25 + l1 * 5 + l2


def _sc_prep_hist():
    mesh = plsc.VectorSubcoreMesh(**_SC_MESH)

    @functools.partial(
        pl.kernel, mesh=mesh, **_SC_PARAMS,
        out_type=(jax.ShapeDtypeStruct((NW, 128), jnp.int32),
                  jax.ShapeDtypeStruct((NW, NPAD), jnp.float32)),
        scratch_types=[
            pltpu.VMEM((EW * 3,), jnp.float32),
            pltpu.VMEM((EW,), jnp.int32),
            pltpu.VMEM((EW,), jnp.float32),
            pltpu.VMEM((EW,), jnp.float32),
            pltpu.VMEM((EW,), jnp.float32),
            pltpu.VMEM((EW,), jnp.int32),
            pltpu.VMEM((128,), jnp.int32),
            pltpu.VMEM((NPAD,), jnp.float32),
        ],
    )
    def k(ps_hbm, dst_hbm, hist_hbm, degp_hbm,
          ps_v, dst_v, f0_v, f1_v, f2_v, lb_v, hist_v, degp_v):
        wid = lax.axis_index("s") * NC + lax.axis_index("c")
        e0 = wid * EW
        iota = lax.broadcasted_iota(jnp.int32, (LANES,), 0)
        pltpu.sync_copy(ps_hbm.at[pl.ds(e0 * 3, EW * 3)], ps_v)
        pltpu.sync_copy(dst_hbm.at[pl.ds(e0, EW)], dst_v)

        @plsc.parallel_loop(0, 128, LANES)
        def z0(t):
            hist_v[pl.ds(t, LANES)] = jnp.zeros((LANES,), jnp.int32)

        @plsc.parallel_loop(0, NPAD, LANES, unroll=8)
        def z1(t):
            degp_v[pl.ds(t, LANES)] = jnp.zeros((LANES,), jnp.float32)

        _spline_loop_a(ps_v, f0_v, f1_v, f2_v, lb_v, iota)
        ones_f = jnp.ones((LANES,), jnp.float32)

        @plsc.parallel_loop(0, EW, LANES)
        def lb(t):
            kb = lb_v[pl.ds(t, LANES)]
            ev = (t + iota + e0) < E
            dd = dst_v[pl.ds(t, LANES)]
            plsc.addupdate_scatter(degp_v, [dd], ones_f, mask=ev)
            for c in range(8):
                kc = kb + OFFC[c]
                cnts, last = plsc.scan_count(kc)
                plsc.addupdate_scatter(hist_v, [kc], cnts, mask=last)

        pltpu.sync_copy(hist_v, hist_hbm.at[wid])
        pltpu.sync_copy(degp_v, degp_hbm.at[wid])
    return k


def _tc_prep_mid():
    def body(hist_ref, degp_ref, base_ref, blockk_ref, deg_ref):
        h = hist_ref[...].astype(jnp.float32)
        tot = jnp.sum(h, axis=0, keepdims=True)
        padded = jnp.floor((tot + (B - 1)) * (1.0 / B)).astype(jnp.float32)
        padded = padded * B
        r128 = lax.broadcasted_iota(jnp.int32, (128, 128), 0)
        c128 = lax.broadcasted_iota(jnp.int32, (128, 128), 1)
        lt128 = (r128 < c128).astype(jnp.float32)
        offs = lax.dot_general(padded, lt128, (((1,), (0,)), ((), ())),
                               preferred_element_type=jnp.float32)
        r32 = lax.broadcasted_iota(jnp.int32, (NW, NW), 0)
        c32 = lax.broadcasted_iota(jnp.int32, (NW, NW), 1)
        lt32 = (r32 > c32).astype(jnp.float32)
        prev = lax.dot_general(lt32, h, (((1,), (0,)), ((), ())),
                               preferred_element_type=jnp.float32)
        base_ref[...] = (offs + prev).astype(jnp.int32)

        jb = (lax.broadcasted_iota(jnp.int32, (NBLK_PAD, 128), 0) * B)
        hit = (offs.astype(jnp.int32) <= jb).astype(jnp.float32)
        nk = jnp.sum(hit, axis=1, keepdims=True).astype(jnp.int32) - 1
        blockk_ref[...] = jnp.clip(nk, 0, KC - 1)

        deg_ref[...] = jnp.maximum(jnp.sum(degp_ref[...], axis=0,
                                           keepdims=True), 1.0)

    return pl.pallas_call(
        body,
        out_shape=(jax.ShapeDtypeStruct((NW, 128), jnp.int32),
                   jax.ShapeDtypeStruct((NBLK_PAD, 1), jnp.int32),
                   jax.ShapeDtypeStruct((1, NPAD), jnp.float32)),
    )


def _sc_prep_pos():
    mesh = plsc.VectorSubcoreMesh(**_SC_MESH)

    @functools.partial(
        pl.kernel, mesh=mesh, **_SC_PARAMS,
        out_type=(jax.ShapeDtypeStruct((NP_PAD,), jnp.int32),
                  jax.ShapeDtypeStruct((NP_PAD,), jnp.float32)),
        scratch_types=[
            pltpu.VMEM((EW * 3,), jnp.float32),
            pltpu.VMEM((EW,), jnp.float32),
            pltpu.VMEM((EW,), jnp.float32),
            pltpu.VMEM((EW,), jnp.float32),
            pltpu.VMEM((EW,), jnp.int32),
            pltpu.VMEM((128,), jnp.int32),
            pltpu.VMEM((8 * EW,), jnp.int32),
            pltpu.VMEM((8 * EW,), jnp.float32),
        ],
    )
    def k(ps_hbm, base_hbm, pos8_hbm, w8_hbm,
          ps_v, f0_v, f1_v, f2_v, lb_v, cnt_v, pos_v, ww_v):
        wid = lax.axis_index("s") * NC + lax.axis_index("c")
        e0 = wid * EW
        iota = lax.broadcasted_iota(jnp.int32, (LANES,), 0)
        pltpu.sync_copy(ps_hbm.at[pl.ds(e0 * 3, EW * 3)], ps_v)
        pltpu.sync_copy(base_hbm.at[pl.ds(wid * 128, 128)], cnt_v)
        _spline_loop_a(ps_v, f0_v, f1_v, f2_v, lb_v, iota)

        def lb(g, carry):
            t = g * LANES
            kb = lb_v[pl.ds(t, LANES)]
            f0 = f0_v[pl.ds(t, LANES)]
            f1 = f1_v[pl.ds(t, LANES)]
            f2 = f2_v[pl.ds(t, LANES)]
            evf = jnp.where((t + iota + e0) < E, 1.0, 0.0)
            for c in range(8):
                kc = kb + OFFC[c]
                wc = _corner_w(c, f0, f1, f2) * evf
                cnts, last = plsc.scan_count(kc)
                basev = plsc.load_gather(cnt_v, [kc])
                pos_v[pl.ds(c * EW + t, LANES)] = basev + cnts - 1
                ww_v[pl.ds(c * EW + t, LANES)] = wc
                plsc.addupdate_scatter(cnt_v, [kc], cnts, mask=last)
            return carry
        lax.fori_loop(0, NGRP, lb, 0)

        for c in range(8):
            pltpu.sync_copy(pos_v.at[pl.ds(c * EW, EW)],
                            pos8_hbm.at[pl.ds(c * E_PAD + e0, EW)])
            pltpu.sync_copy(ww_v.at[pl.ds(c * EW, EW)],
                            w8_hbm.at[pl.ds(c * E_PAD + e0, EW)])
    return k


def _sc_prep_scatter():
    mesh = plsc.VectorSubcoreMesh(**_SC_MESH)
    n_ch = PT // CH2
    rows = CH2 // 128

    @functools.partial(
        pl.kernel, mesh=mesh, **_SC_PARAMS,
        out_type=(jax.ShapeDtypeStruct((P_CAP,), jnp.int32),
                  jax.ShapeDtypeStruct((P_CAP,), jnp.int32),
                  jax.ShapeDtypeStruct((P_CAP,), jnp.float32)),
        scratch_types=[
            pltpu.VMEM((CH2,), jnp.int32),
            pltpu.VMEM((CH2,), jnp.int32),
            pltpu.VMEM((CH2,), jnp.int32),
            pltpu.VMEM((CH2,), jnp.float32),
            pltpu.VMEM((CH2,), jnp.int32),
            pltpu.VMEM_SHARED((HALF + 8,), jnp.int32),
            pltpu.VMEM_SHARED((HALF + 8,), jnp.int32),
            pltpu.VMEM_SHARED((HALF + 8,), jnp.float32),
        ],
    )
    def k(pos8_hbm, w8_hbm, src_hbm, dst_hbm, sinit_hbm, winit_hbm,
          sout_hbm, dout_hbm, wout_hbm,
          pos_v, sv_v, dv_v, wv_v, lidx_v, s_sp, d_sp, w_sp):
        ci = lax.axis_index("c")
        si = lax.axis_index("s")
        gbase = ci * HALF + si * HALF16
        lbase = si * HALF16
        pltpu.sync_copy(sinit_hbm.at[pl.ds(gbase, HALF16)],
                        s_sp.at[pl.ds(lbase, HALF16)])
        pltpu.sync_copy(sinit_hbm.at[pl.ds(gbase, HALF16)],
                        d_sp.at[pl.ds(lbase, HALF16)])
        pltpu.sync_copy(winit_hbm.at[pl.ds(gbase, HALF16)],
                        w_sp.at[pl.ds(lbase, HALF16)])
        plsc.subcore_barrier()

        iota = lax.broadcasted_iota(jnp.int32, (LANES,), 0)
        half_lo = ci * HALF
        for j in range(n_ch):
            pc = si * PT + j * CH2
            ec = pc - (pc // E_PAD) * E_PAD
            pltpu.sync_copy(pos8_hbm.at[pl.ds(pc, CH2)], pos_v)
            pltpu.sync_copy(w8_hbm.at[pl.ds(pc, CH2)], wv_v)
            pltpu.sync_copy(src_hbm.at[pl.ds(ec, CH2)], sv_v)
            pltpu.sync_copy(dst_hbm.at[pl.ds(ec, CH2)], dv_v)

            @plsc.parallel_loop(0, CH2, LANES, unroll=4)
            def pb(t):
                p = pos_v[pl.ds(t, LANES)]
                inh = (p >= half_lo) & (p < half_lo + HALF)
                lidx_v[pl.ds(t, LANES)] = jnp.where(inh, p - half_lo, HALF)

            pltpu.sync_copy(sv_v, s_sp.at[lidx_v])
            pltpu.sync_copy(dv_v, d_sp.at[lidx_v])
            pltpu.sync_copy(wv_v, w_sp.at[lidx_v])
        plsc.subcore_barrier()
        pltpu.sync_copy(s_sp.at[pl.ds(lbase, HALF16)],
                        sout_hbm.at[pl.ds(gbase, HALF16)])
        pltpu.sync_copy(d_sp.at[pl.ds(lbase, HALF16)],
                        dout_hbm.at[pl.ds(gbase, HALF16)])
        pltpu.sync_copy(w_sp.at[pl.ds(lbase, HALF16)],
                        wout_hbm.at[pl.ds(gbase, HALF16)])
    return k


def _prepare_pairs_pallas(pseudo, edge_index):
    ps_flat = jnp.zeros((E_PAD * 3,), jnp.float32).at[:E * 3].set(
        pseudo.reshape(-1))
    src_pad = jnp.zeros((E_PAD,), jnp.int32).at[:E].set(edge_index[0])
    dst_pad = jnp.zeros((E_PAD,), jnp.int32).at[:E].set(edge_index[1])

    hist, degp = _sc_prep_hist()(ps_flat, dst_pad)
    base, blockk, deg = _tc_prep_mid()(hist, degp)
    pos8, w8 = _sc_prep_pos()(ps_flat, base.reshape(-1))
    sinit = jnp.arange(P_CAP, dtype=jnp.int32) % N
    winit = jnp.zeros((P_CAP,), jnp.float32)
    s_sorted, d_sorted, w_sorted = _sc_prep_scatter()(
        pos8, w8, src_pad, dst_pad, sinit, winit)
    block_k = blockk.reshape(-1)[:NBLK]
    return s_sorted, d_sorted, w_sorted, block_k, deg


def _prepare_pairs(pseudo, edge_index):
    src = edge_index[0]
    dst = edge_index[1]
    u = pseudo * (KS - 1)
    lo_f = jnp.clip(jnp.floor(u), 0.0, KS - 2)
    frac = u - lo_f
    lo = lo_f.astype(jnp.int32)

    ws, idxs = [], []
    for bits in range(8):
        w = jnp.ones((E,), jnp.float32)
        idx = jnp.zeros((E,), jnp.int32)
        for d in range(3):
            sel = (bits >> d) & 1
            w = w * (frac[:, d] if sel else 1.0 - frac[:, d])
            idx = idx * KS + (lo[:, d] + sel)
        ws.append(w)
        idxs.append(idx)
    w8 = jnp.concatenate(ws)
    k8 = jnp.concatenate(idxs)
    s8 = jnp.tile(src, 8)
    d8 = jnp.tile(dst, 8)

    counts = jnp.zeros((KC,), jnp.int32).at[k8].add(1)
    padded = ((counts + (B - 1)) // B) * B
    offs = jnp.concatenate([jnp.zeros((1,), jnp.int32),
                            jnp.cumsum(padded)[:-1].astype(jnp.int32)])
    cum_counts = jnp.concatenate([jnp.zeros((1,), jnp.int32),
                                  jnp.cumsum(counts)[:-1].astype(jnp.int32)])

    order = jnp.argsort(k8)
    k_sorted = k8[order]
    rank = jnp.arange(NP, dtype=jnp.int32) - cum_counts[k_sorted]
    pos = offs[k_sorted] + rank

    fill = (jnp.arange(P_CAP, dtype=jnp.int32) % N)
    s_sorted = fill.at[pos].set(s8[order])
    d_sorted = fill.at[pos].set(d8[order])
    w_sorted = jnp.zeros((P_CAP,), jnp.float32).at[pos].set(w8[order])

    blk_start = jnp.arange(NBLK, dtype=jnp.int32) * B
    block_k = jnp.sum(offs[None, :] <= blk_start[:, None], axis=1).astype(jnp.int32) - 1

    deg = jnp.zeros((N,), jnp.float32).at[dst].add(1.0)
    deg = jnp.maximum(deg, 1.0)
    deg = jnp.concatenate([deg, jnp.ones((NPAD - N,), jnp.float32)])
    return s_sorted, d_sorted, w_sorted, block_k, deg.reshape(1, NPAD)


def _sc_gather(din):
    mesh = plsc.VectorSubcoreMesh(core_axis_name="c", subcore_axis_name="s",
                                  num_cores=NC, num_subcores=NS)
    n_chunks = P_CAP // CHUNK

    if din >= NW:
        R = din // NW

        @functools.partial(
            pl.kernel, mesh=mesh,
            compiler_params=pltpu.CompilerParams(needs_layout_passes=False),
            out_type=jax.ShapeDtypeStruct((din, P_CAP), jnp.float32),
            scratch_types=[
                pltpu.VMEM((R * NPAD,), jnp.float32),
                pltpu.VMEM((CHUNK,), jnp.int32),
                pltpu.VMEM((R, CHUNK), jnp.float32),
            ],
        )
        def k(xt_hbm, s_hbm, xgt_hbm, xrow_v, sidx_v, out_v):
            wid = lax.axis_index("s") * NC + lax.axis_index("c")
            row0 = wid * R
            for r in range(R):
                pltpu.sync_copy(xt_hbm.at[row0 + r],
                                xrow_v.at[pl.ds(r * NPAD, NPAD)])

            def chunk_body(g, _):
                base = g * CHUNK
                pltpu.sync_copy(s_hbm.at[pl.ds(base, CHUNK)], sidx_v)

                @plsc.parallel_loop(0, CHUNK, LANES, unroll=8)
                def body(t):
                    idx = sidx_v[pl.ds(t, LANES)]
                    for r in range(R):
                        vals = plsc.load_gather(
                            xrow_v, [idx + jnp.int32(r * NPAD)])
                        out_v[r, pl.ds(t, LANES)] = vals
                pltpu.sync_copy(out_v, xgt_hbm.at[pl.ds(row0, R), pl.ds(base, CHUNK)])
                return 0
            lax.fori_loop(0, n_chunks, chunk_body, 0)
        return k

    per_w = P_CAP // NW

    @functools.partial(
        pl.kernel, mesh=mesh,
        compiler_params=pltpu.CompilerParams(needs_layout_passes=False),
        out_type=jax.ShapeDtypeStruct((din, P_CAP), jnp.float32),
        scratch_types=[
            pltpu.VMEM((NPAD,), jnp.float32),
            pltpu.VMEM((per_w,), jnp.int32),
            pltpu.VMEM((per_w,), jnp.float32),
        ],
    )
    def k1(xt_hbm, s_hbm, xgt_hbm, xrow_v, sidx_v, out_v):
        wid = lax.axis_index("s") * NC + lax.axis_index("c")
        base = wid * per_w
        pltpu.sync_copy(xt_hbm.at[0], xrow_v)
        pltpu.sync_copy(s_hbm.at[pl.ds(base, per_w)], sidx_v)

        @plsc.parallel_loop(0, per_w, LANES, unroll=8)
        def body(t):
            idx = sidx_v[pl.ds(t, LANES)]
            vals = plsc.load_gather(xrow_v, [idx])
            out_v[pl.ds(t, LANES)] = vals
        pltpu.sync_copy(out_v, xgt_hbm.at[0, pl.ds(base, per_w)])
    return k1


def _sc_scatter(dout):
    mesh = plsc.VectorSubcoreMesh(core_axis_name="c", subcore_axis_name="s",
                                  num_cores=NC, num_subcores=NS)
    n_chunks = P_CAP // CHUNK
    R = dout // NW
    assert R >= 1

    @functools.partial(
        pl.kernel, mesh=mesh,
        compiler_params=pltpu.CompilerParams(needs_layout_passes=False),
        out_type=jax.ShapeDtypeStruct((dout, NPAD), jnp.float32),
        scratch_types=[
            pltpu.VMEM((R * NPAD,), jnp.float32),
            pltpu.VMEM((CHUNK,), jnp.int32),
            pltpu.VMEM((R, CHUNK), jnp.float32),
        ],
    )
    def k(mt_hbm, d_hbm, numt_hbm, acc_v, didx_v, m_v):
        wid = lax.axis_index("s") * NC + lax.axis_index("c")
        row0 = wid * R

        @plsc.parallel_loop(0, R * NPAD, LANES, unroll=8)
        def zero_body(t):
            acc_v[pl.ds(t, LANES)] = jnp.zeros((LANES,), jnp.float32)

        def chunk_body(g, _):
            base = g * CHUNK
            pltpu.sync_copy(d_hbm.at[pl.ds(base, CHUNK)], didx_v)
            pltpu.sync_copy(mt_hbm.at[pl.ds(row0, R), pl.ds(base, CHUNK)], m_v)

            @plsc.parallel_loop(0, CHUNK, LANES, unroll=8)
            def body(t):
                idx = didx_v[pl.ds(t, LANES)]
                for r in range(R):
                    vals = m_v[r, pl.ds(t, LANES)]
                    plsc.addupdate_scatter(acc_v, [idx + jnp.int32(r * NPAD)], vals)
            return 0
        lax.fori_loop(0, n_chunks, chunk_body, 0)
        for r in range(R):
            pltpu.sync_copy(acc_v.at[pl.ds(r * NPAD, NPAD)], numt_hbm.at[row0 + r])
    return k


def _tc_matmul(din, dout):
    def body(block_k_ref, xg_ref, w_ref, wgt_ref, out_ref):
        wk = w_ref[...]
        xg = xg_ref[...]
        m = lax.dot_general(wk, xg, (((0,), (0,)), ((), ())),
                            preferred_element_type=jnp.float32)
        out_ref[...] = m * wgt_ref[...]

    grid_spec = pltpu.PrefetchScalarGridSpec(
        num_scalar_prefetch=1,
        grid=(NBLK,),
        in_specs=[
            pl.BlockSpec((din, B), lambda j, bk: (0, j)),
            pl.BlockSpec((None, din, dout), lambda j, bk: (bk[j], 0, 0)),
            pl.BlockSpec((None, 1, B), lambda j, bk: (j, 0, 0)),
        ],
        out_specs=pl.BlockSpec((dout, B), lambda j, bk: (0, j)),
    )
    return pl.pallas_call(
        body, grid_spec=grid_spec,
        out_shape=jax.ShapeDtypeStruct((dout, P_CAP), jnp.float32),
    )


TN = 768


def _tc_epilogue(din, dout):
    def body(num_ref, deg_ref, xt_ref, wr_ref, b_ref, out_ref):
        agg = num_ref[...] / deg_ref[...]
        root = lax.dot_general(wr_ref[...], xt_ref[...], (((0,), (0,)), ((), ())),
                               preferred_element_type=jnp.float32)
        h = agg + root + b_ref[...]
        out_ref[...] = jnp.where(h > 0, h, jnp.exp(h) - 1.0)

    return pl.pallas_call(
        body,
        grid=(NPAD // TN,),
        in_specs=[
            pl.BlockSpec((dout, TN), lambda j: (0, j)),
            pl.BlockSpec((1, TN), lambda j: (0, j)),
            pl.BlockSpec((din, TN), lambda j: (0, j)),
            pl.BlockSpec((din, dout), lambda j: (0, 0)),
            pl.BlockSpec((dout, 1), lambda j: (0, 0)),
        ],
        out_specs=pl.BlockSpec((dout, TN), lambda j: (0, j)),
        out_shape=jax.ShapeDtypeStruct((dout, NPAD), jnp.float32),
    )


def _tc_lin1():
    def body(xt_ref, w_ref, b_ref, out_ref):
        h = lax.dot_general(w_ref[...], xt_ref[...], (((0,), (0,)), ((), ())),
                            preferred_element_type=jnp.float32) + b_ref[...]
        out_ref[...] = jnp.where(h > 0, h, jnp.exp(h) - 1.0)

    return pl.pallas_call(
        body,
        grid=(NPAD // TN,),
        in_specs=[
            pl.BlockSpec((64, TN), lambda j: (0, j)),
            pl.BlockSpec((64, 256), lambda j: (0, 0)),
            pl.BlockSpec((256, 1), lambda j: (0, 0)),
        ],
        out_specs=pl.BlockSpec((256, TN), lambda j: (0, j)),
        out_shape=jax.ShapeDtypeStruct((256, NPAD), jnp.float32),
    )


TR = 256


def _tc_head():
    def body(h_ref, w_ref, b_ref, out_ref):
        logits = lax.dot_general(h_ref[...], w_ref[...], (((0,), (0,)), ((), ())),
                                 preferred_element_type=jnp.float32)
        logits = logits + b_ref[...]
        m = jnp.max(logits, axis=1, keepdims=True)
        lse = m + jnp.log(jnp.sum(jnp.exp(logits - m), axis=1, keepdims=True))
        out_ref[...] = logits - lse

    nblk = (N + TR - 1) // TR
    return pl.pallas_call(
        body,
        grid=(nblk,),
        in_specs=[
            pl.BlockSpec((256, TR), lambda j: (0, j)),
            pl.BlockSpec((256, N), lambda j: (0, 0)),
            pl.BlockSpec((1, N), lambda j: (0, 0)),
        ],
        out_specs=pl.BlockSpec((TR, N), lambda j: (j, 0)),
        out_shape=jax.ShapeDtypeStruct((N, N), jnp.float32),
    )


def kernel(x, pseudo, edge_index, W1, Wr1, b1, W2, Wr2, b2, W3, Wr3, b3,
           W4, Wr4, b4, W5, Wr5, b5, W6, Wr6, b6, lin1_W, lin1_b, lin2_W, lin2_b):
    s_sorted, d_sorted, w_sorted, block_k, deg = _prepare_pairs_pallas(
        pseudo, edge_index)
    w_blk = w_sorted.reshape(NBLK, 1, B)

    xt = jnp.zeros((1, NPAD), jnp.float32).at[:, :N].set(x.T)
    convs = [(W1, Wr1, b1), (W2, Wr2, b2), (W3, Wr3, b3),
             (W4, Wr4, b4), (W5, Wr5, b5), (W6, Wr6, b6)]
    for (W, Wr, b) in convs:
        din, dout = W.shape[1], W.shape[2]
        xgt = _sc_gather(din)(xt, s_sorted)
        mt = _tc_matmul(din, dout)(block_k, xgt, W, w_blk)
        numt = _sc_scatter(dout)(mt, d_sorted)
        xt = _tc_epilogue(din, dout)(numt, deg, xt, Wr, b.reshape(dout, 1))

    h1t = _tc_lin1()(xt, lin1_W, lin1_b.reshape(256, 1))
    out = _tc_head()(h1t, lin2_W, lin2_b.reshape(1, N))
    return out


# speedup vs baseline: 3.3739x; 1.1150x over previous
import functools

import jax
import jax.numpy as jnp
from jax import lax
from jax.experimental import pallas as pl
from jax.experimental.pallas import tpu as pltpu
from jax.experimental.pallas import tpu_sc as plsc

N = 6890
E = 41340
KS = 5
KC = 125
NPAD = 6912
B = 256
NP = 8 * E
CHUNK = 12288
P_CAP = 368640
NBLK = P_CAP // B
NC, NS, LANES = 2, 16, 16
NW = NC * NS


E_PAD = 41472
EW = E_PAD // NW
NGRP = EW // LANES
NP_PAD = 8 * E_PAD
PT = NP_PAD // NS
CH2 = 6912
HALF = P_CAP // 2
HALF16 = HALF // NS
NBLK_PAD = 1536
OFFC = [(c & 1) * 25 + ((c >> 1) & 1) * 5 + ((c >> 2) & 1) for c in range(8)]

_SC_MESH = dict(core_axis_name="c", subcore_axis_name="s",
                num_cores=NC, num_subcores=NS)
_SC_PARAMS = dict(compiler_params=pltpu.CompilerParams(needs_layout_passes=False))


def _corner_w(c, f0, f1, f2):
    t0 = f0 if (c & 1) else 1.0 - f0
    t1 = f1 if ((c >> 1) & 1) else 1.0 - f1
    t2 = f2 if ((c >> 2) & 1) else 1.0 - f2
    return t0 * t1 * t2


def _spline_loop_a(ps_v, f0_v, f1_v, f2_v, lb_v, iota):
    @plsc.parallel_loop(0, EW, LANES, unroll=4)
    def la(t):
        idx3 = (t + iota) * 3
        u0 = plsc.load_gather(ps_v, [idx3]) * (KS - 1.0)
        u1 = plsc.load_gather(ps_v, [idx3 + 1]) * (KS - 1.0)
        u2 = plsc.load_gather(ps_v, [idx3 + 2]) * (KS - 1.0)
        l0 = jnp.minimum(u0.astype(jnp.int32), KS - 2)
        l1 = jnp.minimum(u1.astype(jnp.int32), KS - 2)
        l2 = jnp.minimum(u2.astype(jnp.int32), KS - 2)
        f0_v[pl.ds(t, LANES)] = u0 - l0.astype(jnp.float32)
        f1_v[pl.ds(t, LANES)] = u1 - l1.astype(jnp.float32)
        f2_v[pl.ds(t, LANES)] = u2 - l2.astype(jnp.float32)
        lb_v[pl.ds(t, LANES)] = l0 * 25 + l1 * 5 + l2


def _sc_prep_hist():
    mesh = plsc.VectorSubcoreMesh(**_SC_MESH)

    @functools.partial(
        pl.kernel, mesh=mesh, **_SC_PARAMS,
        out_type=(jax.ShapeDtypeStruct((NW, 128), jnp.int32),
                  jax.ShapeDtypeStruct((NW, NPAD), jnp.float32)),
        scratch_types=[
            pltpu.VMEM((EW * 3,), jnp.float32),
            pltpu.VMEM((EW,), jnp.int32),
            pltpu.VMEM((EW,), jnp.float32),
            pltpu.VMEM((EW,), jnp.float32),
            pltpu.VMEM((EW,), jnp.float32),
            pltpu.VMEM((EW,), jnp.int32),
            pltpu.VMEM((128,), jnp.int32),
            pltpu.VMEM((NPAD,), jnp.float32),
        ],
    )
    def k(ps_hbm, dst_hbm, hist_hbm, degp_hbm,
          ps_v, dst_v, f0_v, f1_v, f2_v, lb_v, hist_v, degp_v):
        wid = lax.axis_index("s") * NC + lax.axis_index("c")
        e0 = wid * EW
        iota = lax.broadcasted_iota(jnp.int32, (LANES,), 0)
        pltpu.sync_copy(ps_hbm.at[pl.ds(e0 * 3, EW * 3)], ps_v)
        pltpu.sync_copy(dst_hbm.at[pl.ds(e0, EW)], dst_v)

        @plsc.parallel_loop(0, 128, LANES)
        def z0(t):
            hist_v[pl.ds(t, LANES)] = jnp.zeros((LANES,), jnp.int32)

        @plsc.parallel_loop(0, NPAD, LANES, unroll=8)
        def z1(t):
            degp_v[pl.ds(t, LANES)] = jnp.zeros((LANES,), jnp.float32)

        _spline_loop_a(ps_v, f0_v, f1_v, f2_v, lb_v, iota)
        ones_f = jnp.ones((LANES,), jnp.float32)

        @plsc.parallel_loop(0, EW, LANES)
        def lb(t):
            kb = lb_v[pl.ds(t, LANES)]
            ev = (t + iota + e0) < E
            dd = dst_v[pl.ds(t, LANES)]
            plsc.addupdate_scatter(degp_v, [dd], ones_f, mask=ev)
            for c in range(8):
                kc = kb + OFFC[c]
                cnts, last = plsc.scan_count(kc)
                plsc.addupdate_scatter(hist_v, [kc], cnts, mask=last)

        pltpu.sync_copy(hist_v, hist_hbm.at[wid])
        pltpu.sync_copy(degp_v, degp_hbm.at[wid])
    return k


def _tc_prep_mid():
    def body(hist_ref, degp_ref, base_ref, blockk_ref, deg_ref):
        h = hist_ref[...].astype(jnp.float32)
        tot = jnp.sum(h, axis=0, keepdims=True)
        padded = jnp.floor((tot + (B - 1)) * (1.0 / B)).astype(jnp.float32)
        padded = padded * B
        r128 = lax.broadcasted_iota(jnp.int32, (128, 128), 0)
        c128 = lax.broadcasted_iota(jnp.int32, (128, 128), 1)
        lt128 = (r128 < c128).astype(jnp.float32)
        offs = lax.dot_general(padded, lt128, (((1,), (0,)), ((), ())),
                               preferred_element_type=jnp.float32)
        r32 = lax.broadcasted_iota(jnp.int32, (NW, NW), 0)
        c32 = lax.broadcasted_iota(jnp.int32, (NW, NW), 1)
        lt32 = (r32 > c32).astype(jnp.float32)
        prev = lax.dot_general(lt32, h, (((1,), (0,)), ((), ())),
                               preferred_element_type=jnp.float32)
        base_ref[...] = (offs + prev).astype(jnp.int32)

        jb = (lax.broadcasted_iota(jnp.int32, (NBLK_PAD, 128), 0) * B)
        hit = (offs.astype(jnp.int32) <= jb).astype(jnp.float32)
        nk = jnp.sum(hit, axis=1, keepdims=True).astype(jnp.int32) - 1
        blockk_ref[...] = jnp.clip(nk, 0, KC - 1)

        deg_ref[...] = jnp.maximum(jnp.sum(degp_ref[...], axis=0,
                                           keepdims=True), 1.0)

    return pl.pallas_call(
        body,
        out_shape=(jax.ShapeDtypeStruct((NW, 128), jnp.int32),
                   jax.ShapeDtypeStruct((NBLK_PAD, 1), jnp.int32),
                   jax.ShapeDtypeStruct((1, NPAD), jnp.float32)),
    )


def _sc_prep_pos():
    mesh = plsc.VectorSubcoreMesh(**_SC_MESH)

    @functools.partial(
        pl.kernel, mesh=mesh, **_SC_PARAMS,
        out_type=(jax.ShapeDtypeStruct((NP_PAD,), jnp.int32),
                  jax.ShapeDtypeStruct((NP_PAD,), jnp.float32)),
        scratch_types=[
            pltpu.VMEM((EW * 3,), jnp.float32),
            pltpu.VMEM((EW,), jnp.float32),
            pltpu.VMEM((EW,), jnp.float32),
            pltpu.VMEM((EW,), jnp.float32),
            pltpu.VMEM((EW,), jnp.int32),
            pltpu.VMEM((128,), jnp.int32),
            pltpu.VMEM((8 * EW,), jnp.int32),
            pltpu.VMEM((8 * EW,), jnp.float32),
        ],
    )
    def k(ps_hbm, base_hbm, pos8_hbm, w8_hbm,
          ps_v, f0_v, f1_v, f2_v, lb_v, cnt_v, pos_v, ww_v):
        wid = lax.axis_index("s") * NC + lax.axis_index("c")
        e0 = wid * EW
        iota = lax.broadcasted_iota(jnp.int32, (LANES,), 0)
        pltpu.sync_copy(ps_hbm.at[pl.ds(e0 * 3, EW * 3)], ps_v)
        pltpu.sync_copy(base_hbm.at[pl.ds(wid * 128, 128)], cnt_v)
        _spline_loop_a(ps_v, f0_v, f1_v, f2_v, lb_v, iota)

        def lb(g, carry):
            t = g * LANES
            kb = lb_v[pl.ds(t, LANES)]
            f0 = f0_v[pl.ds(t, LANES)]
            f1 = f1_v[pl.ds(t, LANES)]
            f2 = f2_v[pl.ds(t, LANES)]
            evf = jnp.where((t + iota + e0) < E, 1.0, 0.0)
            for c in range(8):
                kc = kb + OFFC[c]
                wc = _corner_w(c, f0, f1, f2) * evf
                cnts, last = plsc.scan_count(kc)
                basev = plsc.load_gather(cnt_v, [kc])
                pos_v[pl.ds(c * EW + t, LANES)] = basev + cnts - 1
                ww_v[pl.ds(c * EW + t, LANES)] = wc
                plsc.addupdate_scatter(cnt_v, [kc], cnts, mask=last)
            return carry
        lax.fori_loop(0, NGRP, lb, 0)

        for c in range(8):
            pltpu.sync_copy(pos_v.at[pl.ds(c * EW, EW)],
                            pos8_hbm.at[pl.ds(c * E_PAD + e0, EW)])
            pltpu.sync_copy(ww_v.at[pl.ds(c * EW, EW)],
                            w8_hbm.at[pl.ds(c * E_PAD + e0, EW)])
    return k


def _sc_prep_scatter():
    mesh = plsc.VectorSubcoreMesh(**_SC_MESH)
    n_ch = PT // CH2
    rows = CH2 // 128

    @functools.partial(
        pl.kernel, mesh=mesh, **_SC_PARAMS,
        out_type=(jax.ShapeDtypeStruct((P_CAP,), jnp.int32),
                  jax.ShapeDtypeStruct((P_CAP,), jnp.int32),
                  jax.ShapeDtypeStruct((P_CAP,), jnp.float32)),
        scratch_types=[
            pltpu.VMEM((CH2,), jnp.int32),
            pltpu.VMEM((CH2,), jnp.int32),
            pltpu.VMEM((CH2,), jnp.int32),
            pltpu.VMEM((CH2,), jnp.float32),
            pltpu.VMEM((CH2,), jnp.int32),
            pltpu.VMEM_SHARED((HALF + 8,), jnp.int32),
            pltpu.VMEM_SHARED((HALF + 8,), jnp.int32),
            pltpu.VMEM_SHARED((HALF + 8,), jnp.float32),
        ],
    )
    def k(pos8_hbm, w8_hbm, src_hbm, dst_hbm, sinit_hbm, winit_hbm,
          sout_hbm, dout_hbm, wout_hbm,
          pos_v, sv_v, dv_v, wv_v, lidx_v, s_sp, d_sp, w_sp):
        ci = lax.axis_index("c")
        si = lax.axis_index("s")
        gbase = ci * HALF + si * HALF16
        lbase = si * HALF16
        pltpu.sync_copy(sinit_hbm.at[pl.ds(gbase, HALF16)],
                        s_sp.at[pl.ds(lbase, HALF16)])
        pltpu.sync_copy(sinit_hbm.at[pl.ds(gbase, HALF16)],
                        d_sp.at[pl.ds(lbase, HALF16)])
        pltpu.sync_copy(winit_hbm.at[pl.ds(gbase, HALF16)],
                        w_sp.at[pl.ds(lbase, HALF16)])
        plsc.subcore_barrier()

        iota = lax.broadcasted_iota(jnp.int32, (LANES,), 0)
        half_lo = ci * HALF
        for j in range(n_ch):
            pc = si * PT + j * CH2
            ec = pc - (pc // E_PAD) * E_PAD
            pltpu.sync_copy(pos8_hbm.at[pl.ds(pc, CH2)], pos_v)
            pltpu.sync_copy(w8_hbm.at[pl.ds(pc, CH2)], wv_v)
            pltpu.sync_copy(src_hbm.at[pl.ds(ec, CH2)], sv_v)
            pltpu.sync_copy(dst_hbm.at[pl.ds(ec, CH2)], dv_v)

            @plsc.parallel_loop(0, CH2, LANES, unroll=4)
            def pb(t):
                p = pos_v[pl.ds(t, LANES)]
                inh = (p >= half_lo) & (p < half_lo + HALF)
                lidx_v[pl.ds(t, LANES)] = jnp.where(inh, p - half_lo, HALF)

            pltpu.sync_copy(sv_v, s_sp.at[lidx_v])
            pltpu.sync_copy(dv_v, d_sp.at[lidx_v])
            pltpu.sync_copy(wv_v, w_sp.at[lidx_v])
        plsc.subcore_barrier()
        pltpu.sync_copy(s_sp.at[pl.ds(lbase, HALF16)],
                        sout_hbm.at[pl.ds(gbase, HALF16)])
        pltpu.sync_copy(d_sp.at[pl.ds(lbase, HALF16)],
                        dout_hbm.at[pl.ds(gbase, HALF16)])
        pltpu.sync_copy(w_sp.at[pl.ds(lbase, HALF16)],
                        wout_hbm.at[pl.ds(gbase, HALF16)])
    return k


def _prepare_pairs_pallas(pseudo, edge_index):
    ps_flat = jnp.zeros((E_PAD * 3,), jnp.float32).at[:E * 3].set(
        pseudo.reshape(-1))
    src_pad = jnp.zeros((E_PAD,), jnp.int32).at[:E].set(edge_index[0])
    dst_pad = jnp.zeros((E_PAD,), jnp.int32).at[:E].set(edge_index[1])

    hist, degp = _sc_prep_hist()(ps_flat, dst_pad)
    base, blockk, deg = _tc_prep_mid()(hist, degp)
    pos8, w8 = _sc_prep_pos()(ps_flat, base.reshape(-1))
    sinit = jnp.arange(P_CAP, dtype=jnp.int32) % N
    winit = jnp.zeros((P_CAP,), jnp.float32)
    s_sorted, d_sorted, w_sorted = _sc_prep_scatter()(
        pos8, w8, src_pad, dst_pad, sinit, winit)
    block_k = blockk.reshape(-1)[:NBLK]
    return s_sorted, d_sorted, w_sorted, block_k, deg


def _prepare_pairs(pseudo, edge_index):
    src = edge_index[0]
    dst = edge_index[1]
    u = pseudo * (KS - 1)
    lo_f = jnp.clip(jnp.floor(u), 0.0, KS - 2)
    frac = u - lo_f
    lo = lo_f.astype(jnp.int32)

    ws, idxs = [], []
    for bits in range(8):
        w = jnp.ones((E,), jnp.float32)
        idx = jnp.zeros((E,), jnp.int32)
        for d in range(3):
            sel = (bits >> d) & 1
            w = w * (frac[:, d] if sel else 1.0 - frac[:, d])
            idx = idx * KS + (lo[:, d] + sel)
        ws.append(w)
        idxs.append(idx)
    w8 = jnp.concatenate(ws)
    k8 = jnp.concatenate(idxs)
    s8 = jnp.tile(src, 8)
    d8 = jnp.tile(dst, 8)

    counts = jnp.zeros((KC,), jnp.int32).at[k8].add(1)
    padded = ((counts + (B - 1)) // B) * B
    offs = jnp.concatenate([jnp.zeros((1,), jnp.int32),
                            jnp.cumsum(padded)[:-1].astype(jnp.int32)])
    cum_counts = jnp.concatenate([jnp.zeros((1,), jnp.int32),
                                  jnp.cumsum(counts)[:-1].astype(jnp.int32)])

    order = jnp.argsort(k8)
    k_sorted = k8[order]
    rank = jnp.arange(NP, dtype=jnp.int32) - cum_counts[k_sorted]
    pos = offs[k_sorted] + rank

    fill = (jnp.arange(P_CAP, dtype=jnp.int32) % N)
    s_sorted = fill.at[pos].set(s8[order])
    d_sorted = fill.at[pos].set(d8[order])
    w_sorted = jnp.zeros((P_CAP,), jnp.float32).at[pos].set(w8[order])

    blk_start = jnp.arange(NBLK, dtype=jnp.int32) * B
    block_k = jnp.sum(offs[None, :] <= blk_start[:, None], axis=1).astype(jnp.int32) - 1

    deg = jnp.zeros((N,), jnp.float32).at[dst].add(1.0)
    deg = jnp.maximum(deg, 1.0)
    deg = jnp.concatenate([deg, jnp.ones((NPAD - N,), jnp.float32)])
    return s_sorted, d_sorted, w_sorted, block_k, deg.reshape(1, NPAD)


def _sc_gather(din):
    mesh = plsc.VectorSubcoreMesh(core_axis_name="c", subcore_axis_name="s",
                                  num_cores=NC, num_subcores=NS)
    n_chunks = P_CAP // CHUNK

    if din >= NW:
        R = din // NW

        @functools.partial(
            pl.kernel, mesh=mesh,
            compiler_params=pltpu.CompilerParams(needs_layout_passes=False),
            out_type=jax.ShapeDtypeStruct((din, P_CAP), jnp.float32),
            scratch_types=[
                pltpu.VMEM((R * NPAD,), jnp.float32),
                pltpu.VMEM((2, CHUNK), jnp.int32),
                pltpu.VMEM((2, R, CHUNK), jnp.float32),
                pltpu.SemaphoreType.DMA((2,)),
                pltpu.SemaphoreType.DMA((2,)),
            ],
        )
        def k(xt_hbm, s_hbm, xgt_hbm, xrow_v, sidx_v, out_v, sem_in, sem_out):
            wid = lax.axis_index("s") * NC + lax.axis_index("c")
            row0 = wid * R
            for r in range(R):
                pltpu.sync_copy(xt_hbm.at[row0 + r],
                                xrow_v.at[pl.ds(r * NPAD, NPAD)])

            pltpu.async_copy(s_hbm.at[pl.ds(0, CHUNK)], sidx_v.at[0],
                             sem_in.at[0])

            def chunk_body(gg, _):
                for b in range(2):
                    g = gg * 2 + b
                    base = g * CHUNK
                    pltpu.make_async_copy(s_hbm.at[pl.ds(0, CHUNK)],
                                          sidx_v.at[b], sem_in.at[b]).wait()

                    @pl.when(g + 1 < n_chunks)
                    def _():
                        pltpu.async_copy(
                            s_hbm.at[pl.ds((g + 1) * CHUNK, CHUNK)],
                            sidx_v.at[1 - b], sem_in.at[1 - b])

                    @pl.when(g >= 2)
                    def _():
                        pltpu.make_async_copy(
                            out_v.at[b],
                            xgt_hbm.at[pl.ds(row0, R), pl.ds(0, CHUNK)],
                            sem_out.at[b]).wait()

                    @plsc.parallel_loop(0, CHUNK, LANES, unroll=8)
                    def body(t):
                        idx = sidx_v[b, pl.ds(t, LANES)]
                        for r in range(R):
                            vals = plsc.load_gather(
                                xrow_v, [idx + jnp.int32(r * NPAD)])
                            out_v[b, r, pl.ds(t, LANES)] = vals
                    pltpu.async_copy(
                        out_v.at[b],
                        xgt_hbm.at[pl.ds(row0, R), pl.ds(base, CHUNK)],
                        sem_out.at[b])
                return 0
            lax.fori_loop(0, n_chunks // 2, chunk_body, 0)
            for b in range(2):
                pltpu.make_async_copy(
                    out_v.at[b], xgt_hbm.at[pl.ds(row0, R), pl.ds(0, CHUNK)],
                    sem_out.at[b]).wait()
        return k

    per_w = P_CAP // NW

    @functools.partial(
        pl.kernel, mesh=mesh,
        compiler_params=pltpu.CompilerParams(needs_layout_passes=False),
        out_type=jax.ShapeDtypeStruct((din, P_CAP), jnp.float32),
        scratch_types=[
            pltpu.VMEM((NPAD,), jnp.float32),
            pltpu.VMEM((per_w,), jnp.int32),
            pltpu.VMEM((per_w,), jnp.float32),
        ],
    )
    def k1(xt_hbm, s_hbm, xgt_hbm, xrow_v, sidx_v, out_v):
        wid = lax.axis_index("s") * NC + lax.axis_index("c")
        base = wid * per_w
        pltpu.sync_copy(xt_hbm.at[0], xrow_v)
        pltpu.sync_copy(s_hbm.at[pl.ds(base, per_w)], sidx_v)

        @plsc.parallel_loop(0, per_w, LANES, unroll=8)
        def body(t):
            idx = sidx_v[pl.ds(t, LANES)]
            vals = plsc.load_gather(xrow_v, [idx])
            out_v[pl.ds(t, LANES)] = vals
        pltpu.sync_copy(out_v, xgt_hbm.at[0, pl.ds(base, per_w)])
    return k1


def _sc_scatter(dout):
    mesh = plsc.VectorSubcoreMesh(core_axis_name="c", subcore_axis_name="s",
                                  num_cores=NC, num_subcores=NS)
    n_chunks = P_CAP // CHUNK
    R = dout // NW
    assert R >= 1

    @functools.partial(
        pl.kernel, mesh=mesh,
        compiler_params=pltpu.CompilerParams(needs_layout_passes=False),
        out_type=jax.ShapeDtypeStruct((dout, NPAD), jnp.float32),
        scratch_types=[
            pltpu.VMEM((R * NPAD,), jnp.float32),
            pltpu.VMEM((2, CHUNK), jnp.int32),
            pltpu.VMEM((2, R, CHUNK), jnp.float32),
            pltpu.SemaphoreType.DMA((2,)),
            pltpu.SemaphoreType.DMA((2,)),
        ],
    )
    def k(mt_hbm, d_hbm, numt_hbm, acc_v, didx_v, m_v, sem_d, sem_m):
        wid = lax.axis_index("s") * NC + lax.axis_index("c")
        row0 = wid * R

        @plsc.parallel_loop(0, R * NPAD, LANES, unroll=8)
        def zero_body(t):
            acc_v[pl.ds(t, LANES)] = jnp.zeros((LANES,), jnp.float32)

        pltpu.async_copy(d_hbm.at[pl.ds(0, CHUNK)], didx_v.at[0], sem_d.at[0])
        pltpu.async_copy(mt_hbm.at[pl.ds(row0, R), pl.ds(0, CHUNK)],
                         m_v.at[0], sem_m.at[0])

        def chunk_body(gg, _):
            for b in range(2):
                g = gg * 2 + b
                pltpu.make_async_copy(d_hbm.at[pl.ds(0, CHUNK)],
                                      didx_v.at[b], sem_d.at[b]).wait()
                pltpu.make_async_copy(
                    mt_hbm.at[pl.ds(row0, R), pl.ds(0, CHUNK)],
                    m_v.at[b], sem_m.at[b]).wait()

                @pl.when(g + 1 < n_chunks)
                def _():
                    nbase = (g + 1) * CHUNK
                    pltpu.async_copy(d_hbm.at[pl.ds(nbase, CHUNK)],
                                     didx_v.at[1 - b], sem_d.at[1 - b])
                    pltpu.async_copy(
                        mt_hbm.at[pl.ds(row0, R), pl.ds(nbase, CHUNK)],
                        m_v.at[1 - b], sem_m.at[1 - b])

                @plsc.parallel_loop(0, CHUNK, LANES, unroll=8)
                def body(t):
                    idx = didx_v[b, pl.ds(t, LANES)]
                    for r in range(R):
                        vals = m_v[b, r, pl.ds(t, LANES)]
                        plsc.addupdate_scatter(acc_v, [idx + jnp.int32(r * NPAD)], vals)
            return 0
        lax.fori_loop(0, n_chunks // 2, chunk_body, 0)
        for r in range(R):
            pltpu.sync_copy(acc_v.at[pl.ds(r * NPAD, NPAD)], numt_hbm.at[row0 + r])
    return k


def _tc_matmul(din, dout):
    def body(block_k_ref, xg_ref, w_ref, wgt_ref, out_ref):
        wk = w_ref[...]
        xg = xg_ref[...]
        m = lax.dot_general(wk, xg, (((0,), (0,)), ((), ())),
                            preferred_element_type=jnp.float32)
        out_ref[...] = m * wgt_ref[...]

    grid_spec = pltpu.PrefetchScalarGridSpec(
        num_scalar_prefetch=1,
        grid=(NBLK,),
        in_specs=[
            pl.BlockSpec((din, B), lambda j, bk: (0, j)),
            pl.BlockSpec((None, din, dout), lambda j, bk: (bk[j], 0, 0)),
            pl.BlockSpec((None, 1, B), lambda j, bk: (j, 0, 0)),
        ],
        out_specs=pl.BlockSpec((dout, B), lambda j, bk: (0, j)),
    )
    return pl.pallas_call(
        body, grid_spec=grid_spec,
        out_shape=jax.ShapeDtypeStruct((dout, P_CAP), jnp.float32),
    )


TN = 768


def _tc_epilogue(din, dout):
    def body(num_ref, deg_ref, xt_ref, wr_ref, b_ref, out_ref):
        agg = num_ref[...] / deg_ref[...]
        root = lax.dot_general(wr_ref[...], xt_ref[...], (((0,), (0,)), ((), ())),
                               preferred_element_type=jnp.float32)
        h = agg + root + b_ref[...]
        out_ref[...] = jnp.where(h > 0, h, jnp.exp(h) - 1.0)

    return pl.pallas_call(
        body,
        grid=(NPAD // TN,),
        in_specs=[
            pl.BlockSpec((dout, TN), lambda j: (0, j)),
            pl.BlockSpec((1, TN), lambda j: (0, j)),
            pl.BlockSpec((din, TN), lambda j: (0, j)),
            pl.BlockSpec((din, dout), lambda j: (0, 0)),
            pl.BlockSpec((dout, 1), lambda j: (0, 0)),
        ],
        out_specs=pl.BlockSpec((dout, TN), lambda j: (0, j)),
        out_shape=jax.ShapeDtypeStruct((dout, NPAD), jnp.float32),
    )


def _tc_lin1():
    def body(xt_ref, w_ref, b_ref, out_ref):
        h = lax.dot_general(w_ref[...], xt_ref[...], (((0,), (0,)), ((), ())),
                            preferred_element_type=jnp.float32) + b_ref[...]
        out_ref[...] = jnp.where(h > 0, h, jnp.exp(h) - 1.0)

    return pl.pallas_call(
        body,
        grid=(NPAD // TN,),
        in_specs=[
            pl.BlockSpec((64, TN), lambda j: (0, j)),
            pl.BlockSpec((64, 256), lambda j: (0, 0)),
            pl.BlockSpec((256, 1), lambda j: (0, 0)),
        ],
        out_specs=pl.BlockSpec((256, TN), lambda j: (0, j)),
        out_shape=jax.ShapeDtypeStruct((256, NPAD), jnp.float32),
    )


TR = 256


def _tc_head():
    def body(h_ref, w_ref, b_ref, out_ref):
        logits = lax.dot_general(h_ref[...], w_ref[...], (((0,), (0,)), ((), ())),
                                 preferred_element_type=jnp.float32)
        logits = logits + b_ref[...]
        m = jnp.max(logits, axis=1, keepdims=True)
        lse = m + jnp.log(jnp.sum(jnp.exp(logits - m), axis=1, keepdims=True))
        out_ref[...] = logits - lse

    nblk = (N + TR - 1) // TR
    return pl.pallas_call(
        body,
        grid=(nblk,),
        in_specs=[
            pl.BlockSpec((256, TR), lambda j: (0, j)),
            pl.BlockSpec((256, N), lambda j: (0, 0)),
            pl.BlockSpec((1, N), lambda j: (0, 0)),
        ],
        out_specs=pl.BlockSpec((TR, N), lambda j: (j, 0)),
        out_shape=jax.ShapeDtypeStruct((N, N), jnp.float32),
    )


def kernel(x, pseudo, edge_index, W1, Wr1, b1, W2, Wr2, b2, W3, Wr3, b3,
           W4, Wr4, b4, W5, Wr5, b5, W6, Wr6, b6, lin1_W, lin1_b, lin2_W, lin2_b):
    s_sorted, d_sorted, w_sorted, block_k, deg = _prepare_pairs_pallas(
        pseudo, edge_index)
    w_blk = w_sorted.reshape(NBLK, 1, B)

    xt = jnp.zeros((1, NPAD), jnp.float32).at[:, :N].set(x.T)
    convs = [(W1, Wr1, b1), (W2, Wr2, b2), (W3, Wr3, b3),
             (W4, Wr4, b4), (W5, Wr5, b5), (W6, Wr6, b6)]
    for (W, Wr, b) in convs:
        din, dout = W.shape[1], W.shape[2]
        xgt = _sc_gather(din)(xt, s_sorted)
        mt = _tc_matmul(din, dout)(block_k, xgt, W, w_blk)
        numt = _sc_scatter(dout)(mt, d_sorted)
        xt = _tc_epilogue(din, dout)(numt, deg, xt, Wr, b.reshape(dout, 1))

    h1t = _tc_lin1()(xt, lin1_W, lin1_b.reshape(256, 1))
    out = _tc_head()(h1t, lin2_W, lin2_b.reshape(1, N))
    return out


# speedup vs baseline: 3.3976x; 1.0070x over previous
import functools

import jax
import jax.numpy as jnp
from jax import lax
from jax.experimental import pallas as pl
from jax.experimental.pallas import tpu as pltpu
from jax.experimental.pallas import tpu_sc as plsc

N = 6890
E = 41340
KS = 5
KC = 125
NPAD = 6912
B = 256
NP = 8 * E
CHUNK = 12288
P_CAP = 368640
NBLK = P_CAP // B
NC, NS, LANES = 2, 16, 16
NW = NC * NS


E_PAD = 41472
EW = E_PAD // NW
NGRP = EW // LANES
NP_PAD = 8 * E_PAD
PT = NP_PAD // NS
CH2 = 6912
HALF = P_CAP // 2
HALF16 = HALF // NS
NBLK_PAD = 1536
OFFC = [(c & 1) * 25 + ((c >> 1) & 1) * 5 + ((c >> 2) & 1) for c in range(8)]

_SC_MESH = dict(core_axis_name="c", subcore_axis_name="s",
                num_cores=NC, num_subcores=NS)
_SC_PARAMS = dict(compiler_params=pltpu.CompilerParams(needs_layout_passes=False))


def _corner_w(c, f0, f1, f2):
    t0 = f0 if (c & 1) else 1.0 - f0
    t1 = f1 if ((c >> 1) & 1) else 1.0 - f1
    t2 = f2 if ((c >> 2) & 1) else 1.0 - f2
    return t0 * t1 * t2


def _spline_loop_a(ps_v, f0_v, f1_v, f2_v, lb_v, iota):
    @plsc.parallel_loop(0, EW, LANES, unroll=4)
    def la(t):
        idx3 = (t + iota) * 3
        u0 = plsc.load_gather(ps_v, [idx3]) * (KS - 1.0)
        u1 = plsc.load_gather(ps_v, [idx3 + 1]) * (KS - 1.0)
        u2 = plsc.load_gather(ps_v, [idx3 + 2]) * (KS - 1.0)
        l0 = jnp.minimum(u0.astype(jnp.int32), KS - 2)
        l1 = jnp.minimum(u1.astype(jnp.int32), KS - 2)
        l2 = jnp.minimum(u2.astype(jnp.int32), KS - 2)
        f0_v[pl.ds(t, LANES)] = u0 - l0.astype(jnp.float32)
        f1_v[pl.ds(t, LANES)] = u1 - l1.astype(jnp.float32)
        f2_v[pl.ds(t, LANES)] = u2 - l2.astype(jnp.float32)
        lb_v[pl.ds(t, LANES)] = l0 * 25 + l1 * 5 + l2


def _sc_prep_hist():
    mesh = plsc.VectorSubcoreMesh(**_SC_MESH)

    @functools.partial(
        pl.kernel, mesh=mesh, **_SC_PARAMS,
        out_type=(jax.ShapeDtypeStruct((NW, 128), jnp.int32),
                  jax.ShapeDtypeStruct((NW, NPAD), jnp.float32)),
        scratch_types=[
            pltpu.VMEM((EW * 3,), jnp.float32),
            pltpu.VMEM((EW,), jnp.int32),
            pltpu.VMEM((EW,), jnp.float32),
            pltpu.VMEM((EW,), jnp.float32),
            pltpu.VMEM((EW,), jnp.float32),
            pltpu.VMEM((EW,), jnp.int32),
            pltpu.VMEM((128,), jnp.int32),
            pltpu.VMEM((NPAD,), jnp.float32),
        ],
    )
    def k(ps_hbm, dst_hbm, hist_hbm, degp_hbm,
          ps_v, dst_v, f0_v, f1_v, f2_v, lb_v, hist_v, degp_v):
        wid = lax.axis_index("s") * NC + lax.axis_index("c")
        e0 = wid * EW
        iota = lax.broadcasted_iota(jnp.int32, (LANES,), 0)
        pltpu.sync_copy(ps_hbm.at[pl.ds(e0 * 3, EW * 3)], ps_v)
        pltpu.sync_copy(dst_hbm.at[pl.ds(e0, EW)], dst_v)

        @plsc.parallel_loop(0, 128, LANES)
        def z0(t):
            hist_v[pl.ds(t, LANES)] = jnp.zeros((LANES,), jnp.int32)

        @plsc.parallel_loop(0, NPAD, LANES, unroll=8)
        def z1(t):
            degp_v[pl.ds(t, LANES)] = jnp.zeros((LANES,), jnp.float32)

        _spline_loop_a(ps_v, f0_v, f1_v, f2_v, lb_v, iota)
        ones_f = jnp.ones((LANES,), jnp.float32)

        @plsc.parallel_loop(0, EW, LANES)
        def lb(t):
            kb = lb_v[pl.ds(t, LANES)]
            ev = (t + iota + e0) < E
            dd = dst_v[pl.ds(t, LANES)]
            plsc.addupdate_scatter(degp_v, [dd], ones_f, mask=ev)
            for c in range(8):
                kc = kb + OFFC[c]
                cnts, last = plsc.scan_count(kc)
                plsc.addupdate_scatter(hist_v, [kc], cnts, mask=last)

        pltpu.sync_copy(hist_v, hist_hbm.at[wid])
        pltpu.sync_copy(degp_v, degp_hbm.at[wid])
    return k


def _tc_prep_mid():
    def body(hist_ref, degp_ref, base_ref, blockk_ref, deg_ref):
        h = hist_ref[...].astype(jnp.float32)
        tot = jnp.sum(h, axis=0, keepdims=True)
        padded = jnp.floor((tot + (B - 1)) * (1.0 / B)).astype(jnp.float32)
        padded = padded * B
        r128 = lax.broadcasted_iota(jnp.int32, (128, 128), 0)
        c128 = lax.broadcasted_iota(jnp.int32, (128, 128), 1)
        lt128 = (r128 < c128).astype(jnp.float32)
        offs = lax.dot_general(padded, lt128, (((1,), (0,)), ((), ())),
                               preferred_element_type=jnp.float32)
        r32 = lax.broadcasted_iota(jnp.int32, (NW, NW), 0)
        c32 = lax.broadcasted_iota(jnp.int32, (NW, NW), 1)
        lt32 = (r32 > c32).astype(jnp.float32)
        prev = lax.dot_general(lt32, h, (((1,), (0,)), ((), ())),
                               preferred_element_type=jnp.float32)
        base_ref[...] = (offs + prev).astype(jnp.int32)

        jb = (lax.broadcasted_iota(jnp.int32, (NBLK_PAD, 128), 0) * B)
        hit = (offs.astype(jnp.int32) <= jb).astype(jnp.float32)
        nk = jnp.sum(hit, axis=1, keepdims=True).astype(jnp.int32) - 1
        blockk_ref[...] = jnp.clip(nk, 0, KC - 1)

        deg_ref[...] = jnp.maximum(jnp.sum(degp_ref[...], axis=0,
                                           keepdims=True), 1.0)

    return pl.pallas_call(
        body,
        out_shape=(jax.ShapeDtypeStruct((NW, 128), jnp.int32),
                   jax.ShapeDtypeStruct((NBLK_PAD, 1), jnp.int32),
                   jax.ShapeDtypeStruct((1, NPAD), jnp.float32)),
    )


def _sc_prep_pos():
    mesh = plsc.VectorSubcoreMesh(**_SC_MESH)

    @functools.partial(
        pl.kernel, mesh=mesh, **_SC_PARAMS,
        out_type=(jax.ShapeDtypeStruct((NP_PAD,), jnp.int32),
                  jax.ShapeDtypeStruct((NP_PAD,), jnp.float32)),
        scratch_types=[
            pltpu.VMEM((EW * 3,), jnp.float32),
            pltpu.VMEM((EW,), jnp.float32),
            pltpu.VMEM((EW,), jnp.float32),
            pltpu.VMEM((EW,), jnp.float32),
            pltpu.VMEM((EW,), jnp.int32),
            pltpu.VMEM((128,), jnp.int32),
            pltpu.VMEM((8 * EW,), jnp.int32),
            pltpu.VMEM((8 * EW,), jnp.float32),
        ],
    )
    def k(ps_hbm, base_hbm, pos8_hbm, w8_hbm,
          ps_v, f0_v, f1_v, f2_v, lb_v, cnt_v, pos_v, ww_v):
        wid = lax.axis_index("s") * NC + lax.axis_index("c")
        e0 = wid * EW
        iota = lax.broadcasted_iota(jnp.int32, (LANES,), 0)
        pltpu.sync_copy(ps_hbm.at[pl.ds(e0 * 3, EW * 3)], ps_v)
        pltpu.sync_copy(base_hbm.at[pl.ds(wid * 128, 128)], cnt_v)
        _spline_loop_a(ps_v, f0_v, f1_v, f2_v, lb_v, iota)

        def lb(g, carry):
            t = g * LANES
            kb = lb_v[pl.ds(t, LANES)]
            f0 = f0_v[pl.ds(t, LANES)]
            f1 = f1_v[pl.ds(t, LANES)]
            f2 = f2_v[pl.ds(t, LANES)]
            evf = jnp.where((t + iota + e0) < E, 1.0, 0.0)
            for c in range(8):
                kc = kb + OFFC[c]
                wc = _corner_w(c, f0, f1, f2) * evf
                cnts, last = plsc.scan_count(kc)
                basev = plsc.load_gather(cnt_v, [kc])
                pos_v[pl.ds(c * EW + t, LANES)] = basev + cnts - 1
                ww_v[pl.ds(c * EW + t, LANES)] = wc
                plsc.addupdate_scatter(cnt_v, [kc], cnts, mask=last)
            return carry
        lax.fori_loop(0, NGRP, lb, 0)

        for c in range(8):
            pltpu.sync_copy(pos_v.at[pl.ds(c * EW, EW)],
                            pos8_hbm.at[pl.ds(c * E_PAD + e0, EW)])
            pltpu.sync_copy(ww_v.at[pl.ds(c * EW, EW)],
                            w8_hbm.at[pl.ds(c * E_PAD + e0, EW)])
    return k


def _sc_prep_scatter():
    mesh = plsc.VectorSubcoreMesh(**_SC_MESH)
    n_ch = PT // CH2
    rows = CH2 // 128

    @functools.partial(
        pl.kernel, mesh=mesh, **_SC_PARAMS,
        out_type=(jax.ShapeDtypeStruct((P_CAP,), jnp.int32),
                  jax.ShapeDtypeStruct((P_CAP,), jnp.int32),
                  jax.ShapeDtypeStruct((P_CAP,), jnp.float32)),
        scratch_types=[
            pltpu.VMEM((CH2,), jnp.int32),
            pltpu.VMEM((CH2,), jnp.int32),
            pltpu.VMEM((CH2,), jnp.int32),
            pltpu.VMEM((CH2,), jnp.float32),
            pltpu.VMEM((CH2,), jnp.int32),
            pltpu.VMEM_SHARED((HALF + 8,), jnp.int32),
            pltpu.VMEM_SHARED((HALF + 8,), jnp.int32),
            pltpu.VMEM_SHARED((HALF + 8,), jnp.float32),
        ],
    )
    def k(pos8_hbm, w8_hbm, src_hbm, dst_hbm, sinit_hbm, winit_hbm,
          sout_hbm, dout_hbm, wout_hbm,
          pos_v, sv_v, dv_v, wv_v, lidx_v, s_sp, d_sp, w_sp):
        ci = lax.axis_index("c")
        si = lax.axis_index("s")
        gbase = ci * HALF + si * HALF16
        lbase = si * HALF16
        pltpu.sync_copy(sinit_hbm.at[pl.ds(gbase, HALF16)],
                        s_sp.at[pl.ds(lbase, HALF16)])
        pltpu.sync_copy(sinit_hbm.at[pl.ds(gbase, HALF16)],
                        d_sp.at[pl.ds(lbase, HALF16)])
        pltpu.sync_copy(winit_hbm.at[pl.ds(gbase, HALF16)],
                        w_sp.at[pl.ds(lbase, HALF16)])
        plsc.subcore_barrier()

        iota = lax.broadcasted_iota(jnp.int32, (LANES,), 0)
        half_lo = ci * HALF
        for j in range(n_ch):
            pc = si * PT + j * CH2
            ec = pc - (pc // E_PAD) * E_PAD
            pltpu.sync_copy(pos8_hbm.at[pl.ds(pc, CH2)], pos_v)
            pltpu.sync_copy(w8_hbm.at[pl.ds(pc, CH2)], wv_v)
            pltpu.sync_copy(src_hbm.at[pl.ds(ec, CH2)], sv_v)
            pltpu.sync_copy(dst_hbm.at[pl.ds(ec, CH2)], dv_v)

            @plsc.parallel_loop(0, CH2, LANES, unroll=4)
            def pb(t):
                p = pos_v[pl.ds(t, LANES)]
                inh = (p >= half_lo) & (p < half_lo + HALF)
                lidx_v[pl.ds(t, LANES)] = jnp.where(inh, p - half_lo, HALF)

            pltpu.sync_copy(sv_v, s_sp.at[lidx_v])
            pltpu.sync_copy(dv_v, d_sp.at[lidx_v])
            pltpu.sync_copy(wv_v, w_sp.at[lidx_v])
        plsc.subcore_barrier()
        pltpu.sync_copy(s_sp.at[pl.ds(lbase, HALF16)],
                        sout_hbm.at[pl.ds(gbase, HALF16)])
        pltpu.sync_copy(d_sp.at[pl.ds(lbase, HALF16)],
                        dout_hbm.at[pl.ds(gbase, HALF16)])
        pltpu.sync_copy(w_sp.at[pl.ds(lbase, HALF16)],
                        wout_hbm.at[pl.ds(gbase, HALF16)])
    return k


def _prepare_pairs_pallas(pseudo, edge_index):
    ps_flat = jnp.zeros((E_PAD * 3,), jnp.float32).at[:E * 3].set(
        pseudo.reshape(-1))
    src_pad = jnp.zeros((E_PAD,), jnp.int32).at[:E].set(edge_index[0])
    dst_pad = jnp.zeros((E_PAD,), jnp.int32).at[:E].set(edge_index[1])

    hist, degp = _sc_prep_hist()(ps_flat, dst_pad)
    base, blockk, deg = _tc_prep_mid()(hist, degp)
    pos8, w8 = _sc_prep_pos()(ps_flat, base.reshape(-1))
    sinit = jnp.arange(P_CAP, dtype=jnp.int32) % N
    winit = jnp.zeros((P_CAP,), jnp.float32)
    s_sorted, d_sorted, w_sorted = _sc_prep_scatter()(
        pos8, w8, src_pad, dst_pad, sinit, winit)
    block_k = blockk.reshape(-1)[:NBLK]
    return s_sorted, d_sorted, w_sorted, block_k, deg


def _prepare_pairs(pseudo, edge_index):
    src = edge_index[0]
    dst = edge_index[1]
    u = pseudo * (KS - 1)
    lo_f = jnp.clip(jnp.floor(u), 0.0, KS - 2)
    frac = u - lo_f
    lo = lo_f.astype(jnp.int32)

    ws, idxs = [], []
    for bits in range(8):
        w = jnp.ones((E,), jnp.float32)
        idx = jnp.zeros((E,), jnp.int32)
        for d in range(3):
            sel = (bits >> d) & 1
            w = w * (frac[:, d] if sel else 1.0 - frac[:, d])
            idx = idx * KS + (lo[:, d] + sel)
        ws.append(w)
        idxs.append(idx)
    w8 = jnp.concatenate(ws)
    k8 = jnp.concatenate(idxs)
    s8 = jnp.tile(src, 8)
    d8 = jnp.tile(dst, 8)

    counts = jnp.zeros((KC,), jnp.int32).at[k8].add(1)
    padded = ((counts + (B - 1)) // B) * B
    offs = jnp.concatenate([jnp.zeros((1,), jnp.int32),
                            jnp.cumsum(padded)[:-1].astype(jnp.int32)])
    cum_counts = jnp.concatenate([jnp.zeros((1,), jnp.int32),
                                  jnp.cumsum(counts)[:-1].astype(jnp.int32)])

    order = jnp.argsort(k8)
    k_sorted = k8[order]
    rank = jnp.arange(NP, dtype=jnp.int32) - cum_counts[k_sorted]
    pos = offs[k_sorted] + rank

    fill = (jnp.arange(P_CAP, dtype=jnp.int32) % N)
    s_sorted = fill.at[pos].set(s8[order])
    d_sorted = fill.at[pos].set(d8[order])
    w_sorted = jnp.zeros((P_CAP,), jnp.float32).at[pos].set(w8[order])

    blk_start = jnp.arange(NBLK, dtype=jnp.int32) * B
    block_k = jnp.sum(offs[None, :] <= blk_start[:, None], axis=1).astype(jnp.int32) - 1

    deg = jnp.zeros((N,), jnp.float32).at[dst].add(1.0)
    deg = jnp.maximum(deg, 1.0)
    deg = jnp.concatenate([deg, jnp.ones((NPAD - N,), jnp.float32)])
    return s_sorted, d_sorted, w_sorted, block_k, deg.reshape(1, NPAD)


def _sc_gather(din):
    mesh = plsc.VectorSubcoreMesh(core_axis_name="c", subcore_axis_name="s",
                                  num_cores=NC, num_subcores=NS)
    n_chunks = P_CAP // CHUNK

    if din >= NW:
        R = din // NW

        @functools.partial(
            pl.kernel, mesh=mesh,
            compiler_params=pltpu.CompilerParams(needs_layout_passes=False),
            out_type=jax.ShapeDtypeStruct((din, P_CAP), jnp.float32),
            scratch_types=[
                pltpu.VMEM((R * NPAD,), jnp.float32),
                pltpu.VMEM((2, CHUNK), jnp.int32),
                pltpu.VMEM((2, R, CHUNK), jnp.float32),
                pltpu.SemaphoreType.DMA((2,)),
                pltpu.SemaphoreType.DMA((2,)),
            ],
        )
        def k(xt_hbm, s_hbm, xgt_hbm, xrow_v, sidx_v, out_v, sem_in, sem_out):
            wid = lax.axis_index("s") * NC + lax.axis_index("c")
            row0 = wid * R
            for r in range(R):
                pltpu.sync_copy(xt_hbm.at[row0 + r],
                                xrow_v.at[pl.ds(r * NPAD, NPAD)])

            pltpu.async_copy(s_hbm.at[pl.ds(0, CHUNK)], sidx_v.at[0],
                             sem_in.at[0])

            def chunk_body(gg, _):
                for b in range(2):
                    g = gg * 2 + b
                    base = g * CHUNK
                    pltpu.make_async_copy(s_hbm.at[pl.ds(0, CHUNK)],
                                          sidx_v.at[b], sem_in.at[b]).wait()

                    @pl.when(g + 1 < n_chunks)
                    def _():
                        pltpu.async_copy(
                            s_hbm.at[pl.ds((g + 1) * CHUNK, CHUNK)],
                            sidx_v.at[1 - b], sem_in.at[1 - b])

                    @pl.when(g >= 2)
                    def _():
                        pltpu.make_async_copy(
                            out_v.at[b],
                            xgt_hbm.at[pl.ds(row0, R), pl.ds(0, CHUNK)],
                            sem_out.at[b]).wait()

                    @plsc.parallel_loop(0, CHUNK, LANES, unroll=8)
                    def body(t):
                        idx = sidx_v[b, pl.ds(t, LANES)]
                        for r in range(R):
                            vals = plsc.load_gather(
                                xrow_v, [idx + jnp.int32(r * NPAD)])
                            out_v[b, r, pl.ds(t, LANES)] = vals
                    pltpu.async_copy(
                        out_v.at[b],
                        xgt_hbm.at[pl.ds(row0, R), pl.ds(base, CHUNK)],
                        sem_out.at[b])
                return 0
            lax.fori_loop(0, n_chunks // 2, chunk_body, 0)
            for b in range(2):
                pltpu.make_async_copy(
                    out_v.at[b], xgt_hbm.at[pl.ds(row0, R), pl.ds(0, CHUNK)],
                    sem_out.at[b]).wait()
        return k

    per_w = P_CAP // NW

    @functools.partial(
        pl.kernel, mesh=mesh,
        compiler_params=pltpu.CompilerParams(needs_layout_passes=False),
        out_type=jax.ShapeDtypeStruct((din, P_CAP), jnp.float32),
        scratch_types=[
            pltpu.VMEM((NPAD,), jnp.float32),
            pltpu.VMEM((per_w,), jnp.int32),
            pltpu.VMEM((per_w,), jnp.float32),
        ],
    )
    def k1(xt_hbm, s_hbm, xgt_hbm, xrow_v, sidx_v, out_v):
        wid = lax.axis_index("s") * NC + lax.axis_index("c")
        base = wid * per_w
        pltpu.sync_copy(xt_hbm.at[0], xrow_v)
        pltpu.sync_copy(s_hbm.at[pl.ds(base, per_w)], sidx_v)

        @plsc.parallel_loop(0, per_w, LANES, unroll=8)
        def body(t):
            idx = sidx_v[pl.ds(t, LANES)]
            vals = plsc.load_gather(xrow_v, [idx])
            out_v[pl.ds(t, LANES)] = vals
        pltpu.sync_copy(out_v, xgt_hbm.at[0, pl.ds(base, per_w)])
    return k1


def _sc_scatter(dout):
    mesh = plsc.VectorSubcoreMesh(core_axis_name="c", subcore_axis_name="s",
                                  num_cores=NC, num_subcores=NS)
    n_chunks = P_CAP // CHUNK
    R = dout // NW
    assert R >= 1

    @functools.partial(
        pl.kernel, mesh=mesh,
        compiler_params=pltpu.CompilerParams(needs_layout_passes=False),
        out_type=jax.ShapeDtypeStruct((dout, NPAD), jnp.float32),
        scratch_types=[
            pltpu.VMEM((R * NPAD,), jnp.float32),
            pltpu.VMEM((2, CHUNK), jnp.int32),
            pltpu.VMEM((2, R, CHUNK), jnp.float32),
            pltpu.SemaphoreType.DMA((2,)),
            pltpu.SemaphoreType.DMA((2,)),
        ],
    )
    def k(mt_hbm, d_hbm, numt_hbm, acc_v, didx_v, m_v, sem_d, sem_m):
        wid = lax.axis_index("s") * NC + lax.axis_index("c")
        row0 = wid * R

        @plsc.parallel_loop(0, R * NPAD, LANES, unroll=8)
        def zero_body(t):
            acc_v[pl.ds(t, LANES)] = jnp.zeros((LANES,), jnp.float32)

        pltpu.async_copy(d_hbm.at[pl.ds(0, CHUNK)], didx_v.at[0], sem_d.at[0])
        pltpu.async_copy(mt_hbm.at[pl.ds(row0, R), pl.ds(0, CHUNK)],
                         m_v.at[0], sem_m.at[0])

        def chunk_body(gg, _):
            for b in range(2):
                g = gg * 2 + b
                pltpu.make_async_copy(d_hbm.at[pl.ds(0, CHUNK)],
                                      didx_v.at[b], sem_d.at[b]).wait()
                pltpu.make_async_copy(
                    mt_hbm.at[pl.ds(row0, R), pl.ds(0, CHUNK)],
                    m_v.at[b], sem_m.at[b]).wait()

                @pl.when(g + 1 < n_chunks)
                def _():
                    nbase = (g + 1) * CHUNK
                    pltpu.async_copy(d_hbm.at[pl.ds(nbase, CHUNK)],
                                     didx_v.at[1 - b], sem_d.at[1 - b])
                    pltpu.async_copy(
                        mt_hbm.at[pl.ds(row0, R), pl.ds(nbase, CHUNK)],
                        m_v.at[1 - b], sem_m.at[1 - b])

                @plsc.parallel_loop(0, CHUNK, LANES, unroll=8)
                def body(t):
                    idx = didx_v[b, pl.ds(t, LANES)]
                    for r in range(R):
                        vals = m_v[b, r, pl.ds(t, LANES)]
                        plsc.addupdate_scatter(acc_v, [idx + jnp.int32(r * NPAD)], vals)
            return 0
        lax.fori_loop(0, n_chunks // 2, chunk_body, 0)
        for r in range(R):
            pltpu.sync_copy(acc_v.at[pl.ds(r * NPAD, NPAD)], numt_hbm.at[row0 + r])
    return k


def _tc_matmul(din, dout):
    def body(block_k_ref, xg_ref, w_ref, wgt_ref, out_ref):
        wk = w_ref[...].astype(jnp.bfloat16)
        xg = xg_ref[...].astype(jnp.bfloat16)
        m = lax.dot_general(wk, xg, (((0,), (0,)), ((), ())),
                            preferred_element_type=jnp.float32)
        out_ref[...] = m * wgt_ref[...]

    grid_spec = pltpu.PrefetchScalarGridSpec(
        num_scalar_prefetch=1,
        grid=(NBLK,),
        in_specs=[
            pl.BlockSpec((din, B), lambda j, bk: (0, j)),
            pl.BlockSpec((None, din, dout), lambda j, bk: (bk[j], 0, 0)),
            pl.BlockSpec((None, 1, B), lambda j, bk: (j, 0, 0)),
        ],
        out_specs=pl.BlockSpec((dout, B), lambda j, bk: (0, j)),
    )
    return pl.pallas_call(
        body, grid_spec=grid_spec,
        out_shape=jax.ShapeDtypeStruct((dout, P_CAP), jnp.float32),
    )


TN = 768


def _tc_epilogue(din, dout):
    def body(num_ref, deg_ref, xt_ref, wr_ref, b_ref, out_ref):
        agg = num_ref[...] / deg_ref[...]
        root = lax.dot_general(wr_ref[...], xt_ref[...], (((0,), (0,)), ((), ())),
                               preferred_element_type=jnp.float32)
        h = agg + root + b_ref[...]
        out_ref[...] = jnp.where(h > 0, h, jnp.exp(h) - 1.0)

    return pl.pallas_call(
        body,
        grid=(NPAD // TN,),
        in_specs=[
            pl.BlockSpec((dout, TN), lambda j: (0, j)),
            pl.BlockSpec((1, TN), lambda j: (0, j)),
            pl.BlockSpec((din, TN), lambda j: (0, j)),
            pl.BlockSpec((din, dout), lambda j: (0, 0)),
            pl.BlockSpec((dout, 1), lambda j: (0, 0)),
        ],
        out_specs=pl.BlockSpec((dout, TN), lambda j: (0, j)),
        out_shape=jax.ShapeDtypeStruct((dout, NPAD), jnp.float32),
    )


def _tc_lin1():
    def body(xt_ref, w_ref, b_ref, out_ref):
        h = lax.dot_general(w_ref[...], xt_ref[...], (((0,), (0,)), ((), ())),
                            preferred_element_type=jnp.float32) + b_ref[...]
        out_ref[...] = jnp.where(h > 0, h, jnp.exp(h) - 1.0)

    return pl.pallas_call(
        body,
        grid=(NPAD // TN,),
        in_specs=[
            pl.BlockSpec((64, TN), lambda j: (0, j)),
            pl.BlockSpec((64, 256), lambda j: (0, 0)),
            pl.BlockSpec((256, 1), lambda j: (0, 0)),
        ],
        out_specs=pl.BlockSpec((256, TN), lambda j: (0, j)),
        out_shape=jax.ShapeDtypeStruct((256, NPAD), jnp.float32),
    )


TR = 256


def _tc_head():
    def body(h_ref, w_ref, b_ref, out_ref):
        logits = lax.dot_general(h_ref[...].astype(jnp.bfloat16),
                                 w_ref[...].astype(jnp.bfloat16),
                                 (((0,), (0,)), ((), ())),
                                 preferred_element_type=jnp.float32)
        logits = logits + b_ref[...]
        m = jnp.max(logits, axis=1, keepdims=True)
        lse = m + jnp.log(jnp.sum(jnp.exp(logits - m), axis=1, keepdims=True))
        out_ref[...] = logits - lse

    nblk = (N + TR - 1) // TR
    return pl.pallas_call(
        body,
        grid=(nblk,),
        in_specs=[
            pl.BlockSpec((256, TR), lambda j: (0, j)),
            pl.BlockSpec((256, N), lambda j: (0, 0)),
            pl.BlockSpec((1, N), lambda j: (0, 0)),
        ],
        out_specs=pl.BlockSpec((TR, N), lambda j: (j, 0)),
        out_shape=jax.ShapeDtypeStruct((N, N), jnp.float32),
    )


def kernel(x, pseudo, edge_index, W1, Wr1, b1, W2, Wr2, b2, W3, Wr3, b3,
           W4, Wr4, b4, W5, Wr5, b5, W6, Wr6, b6, lin1_W, lin1_b, lin2_W, lin2_b):
    s_sorted, d_sorted, w_sorted, block_k, deg = _prepare_pairs_pallas(
        pseudo, edge_index)
    w_blk = w_sorted.reshape(NBLK, 1, B)

    xt = jnp.zeros((1, NPAD), jnp.float32).at[:, :N].set(x.T)
    convs = [(W1, Wr1, b1), (W2, Wr2, b2), (W3, Wr3, b3),
             (W4, Wr4, b4), (W5, Wr5, b5), (W6, Wr6, b6)]
    for (W, Wr, b) in convs:
        din, dout = W.shape[1], W.shape[2]
        xgt = _sc_gather(din)(xt, s_sorted)
        mt = _tc_matmul(din, dout)(block_k, xgt, W, w_blk)
        numt = _sc_scatter(dout)(mt, d_sorted)
        xt = _tc_epilogue(din, dout)(numt, deg, xt, Wr, b.reshape(dout, 1))

    h1t = _tc_lin1()(xt, lin1_W, lin1_b.reshape(256, 1))
    out = _tc_head()(h1t, lin2_W, lin2_b.reshape(1, N))
    return out


# speedup vs baseline: 7.1721x; 2.1109x over previous
import functools

import jax
import jax.numpy as jnp
from jax import lax
from jax.experimental import pallas as pl
from jax.experimental.pallas import tpu as pltpu
from jax.experimental.pallas import tpu_sc as plsc

N = 6890
E = 41340
KS = 5
KC = 125
NPAD = 6912
B = 256
NP = 8 * E
CHUNK = 12288
P_CAP = 368640
NBLK = P_CAP // B
NC, NS, LANES = 2, 16, 16
NW = NC * NS


E_PAD = 41472
EW = E_PAD // NW
NGRP = EW // LANES
NP_PAD = 8 * E_PAD
PT = NP_PAD // NS
CH2 = 6912
HALF = P_CAP // 2
HALF16 = HALF // NS
NBLK_PAD = 1536
OFFC = [(c & 1) * 25 + ((c >> 1) & 1) * 5 + ((c >> 2) & 1) for c in range(8)]

_SC_MESH = dict(core_axis_name="c", subcore_axis_name="s",
                num_cores=NC, num_subcores=NS)
_SC_PARAMS = dict(compiler_params=pltpu.CompilerParams(needs_layout_passes=False))


def _corner_w(c, f0, f1, f2):
    t0 = f0 if (c & 1) else 1.0 - f0
    t1 = f1 if ((c >> 1) & 1) else 1.0 - f1
    t2 = f2 if ((c >> 2) & 1) else 1.0 - f2
    return t0 * t1 * t2


def _spline_loop_a(ps_v, f0_v, f1_v, f2_v, lb_v, iota):
    @plsc.parallel_loop(0, EW, LANES, unroll=4)
    def la(t):
        idx3 = (t + iota) * 3
        u0 = plsc.load_gather(ps_v, [idx3]) * (KS - 1.0)
        u1 = plsc.load_gather(ps_v, [idx3 + 1]) * (KS - 1.0)
        u2 = plsc.load_gather(ps_v, [idx3 + 2]) * (KS - 1.0)
        l0 = jnp.minimum(u0.astype(jnp.int32), KS - 2)
        l1 = jnp.minimum(u1.astype(jnp.int32), KS - 2)
        l2 = jnp.minimum(u2.astype(jnp.int32), KS - 2)
        f0_v[pl.ds(t, LANES)] = u0 - l0.astype(jnp.float32)
        f1_v[pl.ds(t, LANES)] = u1 - l1.astype(jnp.float32)
        f2_v[pl.ds(t, LANES)] = u2 - l2.astype(jnp.float32)
        lb_v[pl.ds(t, LANES)] = l0 * 25 + l1 * 5 + l2


def _sc_prep_hist():
    mesh = plsc.VectorSubcoreMesh(**_SC_MESH)

    @functools.partial(
        pl.kernel, mesh=mesh, **_SC_PARAMS,
        out_type=(jax.ShapeDtypeStruct((NW, 128), jnp.int32),
                  jax.ShapeDtypeStruct((NW, NPAD), jnp.float32)),
        scratch_types=[
            pltpu.VMEM((EW * 3,), jnp.float32),
            pltpu.VMEM((EW,), jnp.int32),
            pltpu.VMEM((EW,), jnp.float32),
            pltpu.VMEM((EW,), jnp.float32),
            pltpu.VMEM((EW,), jnp.float32),
            pltpu.VMEM((EW,), jnp.int32),
            pltpu.VMEM((128,), jnp.int32),
            pltpu.VMEM((NPAD,), jnp.float32),
        ],
    )
    def k(ps_hbm, dst_hbm, hist_hbm, degp_hbm,
          ps_v, dst_v, f0_v, f1_v, f2_v, lb_v, hist_v, degp_v):
        wid = lax.axis_index("s") * NC + lax.axis_index("c")
        e0 = wid * EW
        iota = lax.broadcasted_iota(jnp.int32, (LANES,), 0)
        pltpu.sync_copy(ps_hbm.at[pl.ds(e0 * 3, EW * 3)], ps_v)
        pltpu.sync_copy(dst_hbm.at[pl.ds(e0, EW)], dst_v)

        @plsc.parallel_loop(0, 128, LANES)
        def z0(t):
            hist_v[pl.ds(t, LANES)] = jnp.zeros((LANES,), jnp.int32)

        @plsc.parallel_loop(0, NPAD, LANES, unroll=8)
        def z1(t):
            degp_v[pl.ds(t, LANES)] = jnp.zeros((LANES,), jnp.float32)

        _spline_loop_a(ps_v, f0_v, f1_v, f2_v, lb_v, iota)
        ones_f = jnp.ones((LANES,), jnp.float32)

        @plsc.parallel_loop(0, EW, LANES)
        def lb(t):
            kb = lb_v[pl.ds(t, LANES)]
            ev = (t + iota + e0) < E
            dd = dst_v[pl.ds(t, LANES)]
            plsc.addupdate_scatter(degp_v, [dd], ones_f, mask=ev)
            for c in range(8):
                kc = kb + OFFC[c]
                cnts, last = plsc.scan_count(kc)
                plsc.addupdate_scatter(hist_v, [kc], cnts, mask=last)

        pltpu.sync_copy(hist_v, hist_hbm.at[wid])
        pltpu.sync_copy(degp_v, degp_hbm.at[wid])
    return k


def _tc_prep_mid():
    def body(hist_ref, degp_ref, base_ref, blockk_ref, deg_ref):
        h = hist_ref[...].astype(jnp.float32)
        tot = jnp.sum(h, axis=0, keepdims=True)
        padded = jnp.floor((tot + (B - 1)) * (1.0 / B)).astype(jnp.float32)
        padded = padded * B
        r128 = lax.broadcasted_iota(jnp.int32, (128, 128), 0)
        c128 = lax.broadcasted_iota(jnp.int32, (128, 128), 1)
        lt128 = (r128 < c128).astype(jnp.float32)
        offs = lax.dot_general(padded, lt128, (((1,), (0,)), ((), ())),
                               preferred_element_type=jnp.float32)
        r32 = lax.broadcasted_iota(jnp.int32, (NW, NW), 0)
        c32 = lax.broadcasted_iota(jnp.int32, (NW, NW), 1)
        lt32 = (r32 > c32).astype(jnp.float32)
        prev = lax.dot_general(lt32, h, (((1,), (0,)), ((), ())),
                               preferred_element_type=jnp.float32)
        base_ref[...] = (offs + prev).astype(jnp.int32)

        jb = (lax.broadcasted_iota(jnp.int32, (NBLK_PAD, 128), 0) * B)
        hit = (offs.astype(jnp.int32) <= jb).astype(jnp.float32)
        nk = jnp.sum(hit, axis=1, keepdims=True).astype(jnp.int32) - 1
        blockk_ref[...] = jnp.clip(nk, 0, KC - 1)

        deg_ref[...] = jnp.maximum(jnp.sum(degp_ref[...], axis=0,
                                           keepdims=True), 1.0)

    return pl.pallas_call(
        body,
        out_shape=(jax.ShapeDtypeStruct((NW, 128), jnp.int32),
                   jax.ShapeDtypeStruct((NBLK_PAD, 1), jnp.int32),
                   jax.ShapeDtypeStruct((1, NPAD), jnp.float32)),
    )


def _sc_prep_pos():
    mesh = plsc.VectorSubcoreMesh(**_SC_MESH)

    @functools.partial(
        pl.kernel, mesh=mesh, **_SC_PARAMS,
        out_type=(jax.ShapeDtypeStruct((NP_PAD,), jnp.int32),
                  jax.ShapeDtypeStruct((NP_PAD,), jnp.float32)),
        scratch_types=[
            pltpu.VMEM((EW * 3,), jnp.float32),
            pltpu.VMEM((EW,), jnp.float32),
            pltpu.VMEM((EW,), jnp.float32),
            pltpu.VMEM((EW,), jnp.float32),
            pltpu.VMEM((EW,), jnp.int32),
            pltpu.VMEM((128,), jnp.int32),
            pltpu.VMEM((8 * EW,), jnp.int32),
            pltpu.VMEM((8 * EW,), jnp.float32),
        ],
    )
    def k(ps_hbm, base_hbm, pos8_hbm, w8_hbm,
          ps_v, f0_v, f1_v, f2_v, lb_v, cnt_v, pos_v, ww_v):
        wid = lax.axis_index("s") * NC + lax.axis_index("c")
        e0 = wid * EW
        iota = lax.broadcasted_iota(jnp.int32, (LANES,), 0)
        pltpu.sync_copy(ps_hbm.at[pl.ds(e0 * 3, EW * 3)], ps_v)
        pltpu.sync_copy(base_hbm.at[pl.ds(wid * 128, 128)], cnt_v)
        _spline_loop_a(ps_v, f0_v, f1_v, f2_v, lb_v, iota)

        def lb(g, carry):
            t = g * LANES
            kb = lb_v[pl.ds(t, LANES)]
            f0 = f0_v[pl.ds(t, LANES)]
            f1 = f1_v[pl.ds(t, LANES)]
            f2 = f2_v[pl.ds(t, LANES)]
            evf = jnp.where((t + iota + e0) < E, 1.0, 0.0)
            for c in range(8):
                kc = kb + OFFC[c]
                wc = _corner_w(c, f0, f1, f2) * evf
                cnts, last = plsc.scan_count(kc)
                basev = plsc.load_gather(cnt_v, [kc])
                pos_v[pl.ds(c * EW + t, LANES)] = basev + cnts - 1
                ww_v[pl.ds(c * EW + t, LANES)] = wc
                plsc.addupdate_scatter(cnt_v, [kc], cnts, mask=last)
            return carry
        lax.fori_loop(0, NGRP, lb, 0)

        for c in range(8):
            pltpu.sync_copy(pos_v.at[pl.ds(c * EW, EW)],
                            pos8_hbm.at[pl.ds(c * E_PAD + e0, EW)])
            pltpu.sync_copy(ww_v.at[pl.ds(c * EW, EW)],
                            w8_hbm.at[pl.ds(c * E_PAD + e0, EW)])
    return k


def _sc_prep_scatter():
    mesh = plsc.VectorSubcoreMesh(**_SC_MESH)
    n_ch = PT // CH2
    rows = CH2 // 128

    @functools.partial(
        pl.kernel, mesh=mesh, **_SC_PARAMS,
        out_type=(jax.ShapeDtypeStruct((P_CAP,), jnp.int32),
                  jax.ShapeDtypeStruct((P_CAP,), jnp.int32),
                  jax.ShapeDtypeStruct((P_CAP,), jnp.float32)),
        scratch_types=[
            pltpu.VMEM((CH2,), jnp.int32),
            pltpu.VMEM((CH2,), jnp.int32),
            pltpu.VMEM((CH2,), jnp.int32),
            pltpu.VMEM((CH2,), jnp.float32),
            pltpu.VMEM((CH2,), jnp.int32),
            pltpu.VMEM_SHARED((HALF + 8,), jnp.int32),
            pltpu.VMEM_SHARED((HALF + 8,), jnp.int32),
            pltpu.VMEM_SHARED((HALF + 8,), jnp.float32),
        ],
    )
    def k(pos8_hbm, w8_hbm, src_hbm, dst_hbm, sinit_hbm, winit_hbm,
          sout_hbm, dout_hbm, wout_hbm,
          pos_v, sv_v, dv_v, wv_v, lidx_v, s_sp, d_sp, w_sp):
        ci = lax.axis_index("c")
        si = lax.axis_index("s")
        gbase = ci * HALF + si * HALF16
        lbase = si * HALF16
        pltpu.sync_copy(sinit_hbm.at[pl.ds(gbase, HALF16)],
                        s_sp.at[pl.ds(lbase, HALF16)])
        pltpu.sync_copy(sinit_hbm.at[pl.ds(gbase, HALF16)],
                        d_sp.at[pl.ds(lbase, HALF16)])
        pltpu.sync_copy(winit_hbm.at[pl.ds(gbase, HALF16)],
                        w_sp.at[pl.ds(lbase, HALF16)])
        plsc.subcore_barrier()

        iota = lax.broadcasted_iota(jnp.int32, (LANES,), 0)
        half_lo = ci * HALF
        for j in range(n_ch):
            pc = si * PT + j * CH2
            ec = pc - (pc // E_PAD) * E_PAD
            pltpu.sync_copy(pos8_hbm.at[pl.ds(pc, CH2)], pos_v)
            pltpu.sync_copy(w8_hbm.at[pl.ds(pc, CH2)], wv_v)
            pltpu.sync_copy(src_hbm.at[pl.ds(ec, CH2)], sv_v)
            pltpu.sync_copy(dst_hbm.at[pl.ds(ec, CH2)], dv_v)

            @plsc.parallel_loop(0, CH2, LANES, unroll=4)
            def pb(t):
                p = pos_v[pl.ds(t, LANES)]
                inh = (p >= half_lo) & (p < half_lo + HALF)
                lidx_v[pl.ds(t, LANES)] = jnp.where(inh, p - half_lo, HALF)

            pltpu.sync_copy(sv_v, s_sp.at[lidx_v])
            pltpu.sync_copy(dv_v, d_sp.at[lidx_v])
            pltpu.sync_copy(wv_v, w_sp.at[lidx_v])
        plsc.subcore_barrier()
        pltpu.sync_copy(s_sp.at[pl.ds(lbase, HALF16)],
                        sout_hbm.at[pl.ds(gbase, HALF16)])
        pltpu.sync_copy(d_sp.at[pl.ds(lbase, HALF16)],
                        dout_hbm.at[pl.ds(gbase, HALF16)])
        pltpu.sync_copy(w_sp.at[pl.ds(lbase, HALF16)],
                        wout_hbm.at[pl.ds(gbase, HALF16)])
    return k


def _prepare_pairs_pallas(pseudo, edge_index):
    ps_flat = jnp.zeros((E_PAD * 3,), jnp.float32).at[:E * 3].set(
        pseudo.reshape(-1))
    src_pad = jnp.zeros((E_PAD,), jnp.int32).at[:E].set(edge_index[0])
    dst_pad = jnp.zeros((E_PAD,), jnp.int32).at[:E].set(edge_index[1])

    hist, degp = _sc_prep_hist()(ps_flat, dst_pad)
    base, blockk, deg = _tc_prep_mid()(hist, degp)
    pos8, w8 = _sc_prep_pos()(ps_flat, base.reshape(-1))
    sinit = jnp.arange(P_CAP, dtype=jnp.int32) % N
    winit = jnp.zeros((P_CAP,), jnp.float32)
    s_sorted, d_sorted, w_sorted = _sc_prep_scatter()(
        pos8, w8, src_pad, dst_pad, sinit, winit)
    block_k = blockk.reshape(-1)[:NBLK]
    return s_sorted, d_sorted, w_sorted, block_k, deg


def _prepare_pairs(pseudo, edge_index):
    src = edge_index[0]
    dst = edge_index[1]
    u = pseudo * (KS - 1)
    lo_f = jnp.clip(jnp.floor(u), 0.0, KS - 2)
    frac = u - lo_f
    lo = lo_f.astype(jnp.int32)

    ws, idxs = [], []
    for bits in range(8):
        w = jnp.ones((E,), jnp.float32)
        idx = jnp.zeros((E,), jnp.int32)
        for d in range(3):
            sel = (bits >> d) & 1
            w = w * (frac[:, d] if sel else 1.0 - frac[:, d])
            idx = idx * KS + (lo[:, d] + sel)
        ws.append(w)
        idxs.append(idx)
    w8 = jnp.concatenate(ws)
    k8 = jnp.concatenate(idxs)
    s8 = jnp.tile(src, 8)
    d8 = jnp.tile(dst, 8)

    counts = jnp.zeros((KC,), jnp.int32).at[k8].add(1)
    padded = ((counts + (B - 1)) // B) * B
    offs = jnp.concatenate([jnp.zeros((1,), jnp.int32),
                            jnp.cumsum(padded)[:-1].astype(jnp.int32)])
    cum_counts = jnp.concatenate([jnp.zeros((1,), jnp.int32),
                                  jnp.cumsum(counts)[:-1].astype(jnp.int32)])

    order = jnp.argsort(k8)
    k_sorted = k8[order]
    rank = jnp.arange(NP, dtype=jnp.int32) - cum_counts[k_sorted]
    pos = offs[k_sorted] + rank

    fill = (jnp.arange(P_CAP, dtype=jnp.int32) % N)
    s_sorted = fill.at[pos].set(s8[order])
    d_sorted = fill.at[pos].set(d8[order])
    w_sorted = jnp.zeros((P_CAP,), jnp.float32).at[pos].set(w8[order])

    blk_start = jnp.arange(NBLK, dtype=jnp.int32) * B
    block_k = jnp.sum(offs[None, :] <= blk_start[:, None], axis=1).astype(jnp.int32) - 1

    deg = jnp.zeros((N,), jnp.float32).at[dst].add(1.0)
    deg = jnp.maximum(deg, 1.0)
    deg = jnp.concatenate([deg, jnp.ones((NPAD - N,), jnp.float32)])
    return s_sorted, d_sorted, w_sorted, block_k, deg.reshape(1, NPAD)


def _sc_gather(din):
    mesh = plsc.VectorSubcoreMesh(core_axis_name="c", subcore_axis_name="s",
                                  num_cores=NC, num_subcores=NS)
    n_chunks = P_CAP // CHUNK

    if din >= NW:
        R = din // NW

        @functools.partial(
            pl.kernel, mesh=mesh,
            compiler_params=pltpu.CompilerParams(needs_layout_passes=False),
            out_type=jax.ShapeDtypeStruct((din, P_CAP), jnp.float32),
            scratch_types=[
                pltpu.VMEM((R * NPAD,), jnp.float32),
                pltpu.VMEM((2, CHUNK), jnp.int32),
                pltpu.VMEM((2, R, CHUNK), jnp.float32),
                pltpu.SemaphoreType.DMA((2,)),
                pltpu.SemaphoreType.DMA((2,)),
            ],
        )
        def k(xt_hbm, s_hbm, xgt_hbm, xrow_v, sidx_v, out_v, sem_in, sem_out):
            wid = lax.axis_index("s") * NC + lax.axis_index("c")
            row0 = wid * R
            for r in range(R):
                pltpu.sync_copy(xt_hbm.at[row0 + r],
                                xrow_v.at[pl.ds(r * NPAD, NPAD)])

            pltpu.async_copy(s_hbm.at[pl.ds(0, CHUNK)], sidx_v.at[0],
                             sem_in.at[0])

            def chunk_body(gg, _):
                for b in range(2):
                    g = gg * 2 + b
                    base = g * CHUNK
                    pltpu.make_async_copy(s_hbm.at[pl.ds(0, CHUNK)],
                                          sidx_v.at[b], sem_in.at[b]).wait()

                    @pl.when(g + 1 < n_chunks)
                    def _():
                        pltpu.async_copy(
                            s_hbm.at[pl.ds((g + 1) * CHUNK, CHUNK)],
                            sidx_v.at[1 - b], sem_in.at[1 - b])

                    @pl.when(g >= 2)
                    def _():
                        pltpu.make_async_copy(
                            out_v.at[b],
                            xgt_hbm.at[pl.ds(row0, R), pl.ds(0, CHUNK)],
                            sem_out.at[b]).wait()

                    @plsc.parallel_loop(0, CHUNK, LANES, unroll=8)
                    def body(t):
                        idx = sidx_v[b, pl.ds(t, LANES)]
                        for r in range(R):
                            vals = plsc.load_gather(
                                xrow_v, [idx + jnp.int32(r * NPAD)])
                            out_v[b, r, pl.ds(t, LANES)] = vals
                    pltpu.async_copy(
                        out_v.at[b],
                        xgt_hbm.at[pl.ds(row0, R), pl.ds(base, CHUNK)],
                        sem_out.at[b])
                return 0
            lax.fori_loop(0, n_chunks // 2, chunk_body, 0)
            for b in range(2):
                pltpu.make_async_copy(
                    out_v.at[b], xgt_hbm.at[pl.ds(row0, R), pl.ds(0, CHUNK)],
                    sem_out.at[b]).wait()
        return k

    per_w = P_CAP // NW

    @functools.partial(
        pl.kernel, mesh=mesh,
        compiler_params=pltpu.CompilerParams(needs_layout_passes=False),
        out_type=jax.ShapeDtypeStruct((din, P_CAP), jnp.float32),
        scratch_types=[
            pltpu.VMEM((NPAD,), jnp.float32),
            pltpu.VMEM((per_w,), jnp.int32),
            pltpu.VMEM((per_w,), jnp.float32),
        ],
    )
    def k1(xt_hbm, s_hbm, xgt_hbm, xrow_v, sidx_v, out_v):
        wid = lax.axis_index("s") * NC + lax.axis_index("c")
        base = wid * per_w
        pltpu.sync_copy(xt_hbm.at[0], xrow_v)
        pltpu.sync_copy(s_hbm.at[pl.ds(base, per_w)], sidx_v)

        @plsc.parallel_loop(0, per_w, LANES, unroll=8)
        def body(t):
            idx = sidx_v[pl.ds(t, LANES)]
            vals = plsc.load_gather(xrow_v, [idx])
            out_v[pl.ds(t, LANES)] = vals
        pltpu.sync_copy(out_v, xgt_hbm.at[0, pl.ds(base, per_w)])
    return k1


def _sc_scatter(dout):
    mesh = plsc.VectorSubcoreMesh(core_axis_name="c", subcore_axis_name="s",
                                  num_cores=NC, num_subcores=NS)
    n_chunks = P_CAP // CHUNK
    R = dout // NW
    assert R >= 1

    @functools.partial(
        pl.kernel, mesh=mesh,
        compiler_params=pltpu.CompilerParams(needs_layout_passes=False),
        out_type=jax.ShapeDtypeStruct((dout, NPAD), jnp.float32),
        scratch_types=[
            pltpu.VMEM((R * NPAD,), jnp.float32),
            pltpu.VMEM((2, CHUNK), jnp.int32),
            pltpu.VMEM((2, R, CHUNK), jnp.float32),
            pltpu.SemaphoreType.DMA((2,)),
            pltpu.SemaphoreType.DMA((2,)),
        ],
    )
    def k(mt_hbm, d_hbm, numt_hbm, acc_v, didx_v, m_v, sem_d, sem_m):
        wid = lax.axis_index("s") * NC + lax.axis_index("c")
        row0 = wid * R

        @plsc.parallel_loop(0, R * NPAD, LANES, unroll=8)
        def zero_body(t):
            acc_v[pl.ds(t, LANES)] = jnp.zeros((LANES,), jnp.float32)

        pltpu.async_copy(d_hbm.at[pl.ds(0, CHUNK)], didx_v.at[0], sem_d.at[0])
        pltpu.async_copy(mt_hbm.at[pl.ds(row0, R), pl.ds(0, CHUNK)],
                         m_v.at[0], sem_m.at[0])

        def chunk_body(gg, _):
            for b in range(2):
                g = gg * 2 + b
                pltpu.make_async_copy(d_hbm.at[pl.ds(0, CHUNK)],
                                      didx_v.at[b], sem_d.at[b]).wait()
                pltpu.make_async_copy(
                    mt_hbm.at[pl.ds(row0, R), pl.ds(0, CHUNK)],
                    m_v.at[b], sem_m.at[b]).wait()

                @pl.when(g + 1 < n_chunks)
                def _():
                    nbase = (g + 1) * CHUNK
                    pltpu.async_copy(d_hbm.at[pl.ds(nbase, CHUNK)],
                                     didx_v.at[1 - b], sem_d.at[1 - b])
                    pltpu.async_copy(
                        mt_hbm.at[pl.ds(row0, R), pl.ds(nbase, CHUNK)],
                        m_v.at[1 - b], sem_m.at[1 - b])

                @plsc.parallel_loop(0, CHUNK, LANES, unroll=8)
                def body(t):
                    idx = didx_v[b, pl.ds(t, LANES)]
                    for r in range(R):
                        vals = m_v[b, r, pl.ds(t, LANES)]
                        plsc.addupdate_scatter(acc_v, [idx + jnp.int32(r * NPAD)], vals)
            return 0
        lax.fori_loop(0, n_chunks // 2, chunk_body, 0)
        for r in range(R):
            pltpu.sync_copy(acc_v.at[pl.ds(r * NPAD, NPAD)], numt_hbm.at[row0 + r])
    return k


def _tc_matmul(din, dout):
    UB = 4

    def body(block_k_ref, xg_ref, *rest):
        w_refs = rest[:UB]
        wgt_ref, out_ref = rest[UB], rest[UB + 1]
        for u in range(UB):
            wk = w_refs[u][...].astype(jnp.bfloat16)
            xg = xg_ref[:, u * B:(u + 1) * B].astype(jnp.bfloat16)
            m = lax.dot_general(wk, xg, (((0,), (0,)), ((), ())),
                                preferred_element_type=jnp.float32)
            out_ref[:, u * B:(u + 1) * B] = m * wgt_ref[:, u * B:(u + 1) * B]

    def mk_wspec(u):
        return pl.BlockSpec((None, din, dout),
                            lambda j, bk, u=u: (bk[j * UB + u], 0, 0))

    grid_spec = pltpu.PrefetchScalarGridSpec(
        num_scalar_prefetch=1,
        grid=(NBLK // UB,),
        in_specs=[
            pl.BlockSpec((din, UB * B), lambda j, bk: (0, j)),
        ] + [mk_wspec(u) for u in range(UB)] + [
            pl.BlockSpec((1, UB * B), lambda j, bk: (0, j)),
        ],
        out_specs=pl.BlockSpec((dout, UB * B), lambda j, bk: (0, j)),
    )
    return pl.pallas_call(
        body, grid_spec=grid_spec,
        out_shape=jax.ShapeDtypeStruct((dout, P_CAP), jnp.float32),
    )


TN = 768


def _tc_epilogue(din, dout):
    def body(num_ref, deg_ref, xt_ref, wr_ref, b_ref, out_ref):
        agg = num_ref[...] / deg_ref[...]
        root = lax.dot_general(wr_ref[...], xt_ref[...], (((0,), (0,)), ((), ())),
                               preferred_element_type=jnp.float32)
        h = agg + root + b_ref[...]
        out_ref[...] = jnp.where(h > 0, h, jnp.exp(h) - 1.0)

    return pl.pallas_call(
        body,
        grid=(NPAD // TN,),
        in_specs=[
            pl.BlockSpec((dout, TN), lambda j: (0, j)),
            pl.BlockSpec((1, TN), lambda j: (0, j)),
            pl.BlockSpec((din, TN), lambda j: (0, j)),
            pl.BlockSpec((din, dout), lambda j: (0, 0)),
            pl.BlockSpec((dout, 1), lambda j: (0, 0)),
        ],
        out_specs=pl.BlockSpec((dout, TN), lambda j: (0, j)),
        out_shape=jax.ShapeDtypeStruct((dout, NPAD), jnp.float32),
    )


def _tc_lin1():
    def body(xt_ref, w_ref, b_ref, out_ref):
        h = lax.dot_general(w_ref[...], xt_ref[...], (((0,), (0,)), ((), ())),
                            preferred_element_type=jnp.float32) + b_ref[...]
        out_ref[...] = jnp.where(h > 0, h, jnp.exp(h) - 1.0)

    return pl.pallas_call(
        body,
        grid=(NPAD // TN,),
        in_specs=[
            pl.BlockSpec((64, TN), lambda j: (0, j)),
            pl.BlockSpec((64, 256), lambda j: (0, 0)),
            pl.BlockSpec((256, 1), lambda j: (0, 0)),
        ],
        out_specs=pl.BlockSpec((256, TN), lambda j: (0, j)),
        out_shape=jax.ShapeDtypeStruct((256, NPAD), jnp.float32),
    )


TR = 256


def _tc_head():
    def body(h_ref, w_ref, b_ref, out_ref):
        logits = lax.dot_general(h_ref[...].astype(jnp.bfloat16),
                                 w_ref[...].astype(jnp.bfloat16),
                                 (((0,), (0,)), ((), ())),
                                 preferred_element_type=jnp.float32)
        logits = logits + b_ref[...]
        m = jnp.max(logits, axis=1, keepdims=True)
        lse = m + jnp.log(jnp.sum(jnp.exp(logits - m), axis=1, keepdims=True))
        out_ref[...] = logits - lse

    nblk = (N + TR - 1) // TR
    return pl.pallas_call(
        body,
        grid=(nblk,),
        in_specs=[
            pl.BlockSpec((256, TR), lambda j: (0, j)),
            pl.BlockSpec((256, N), lambda j: (0, 0)),
            pl.BlockSpec((1, N), lambda j: (0, 0)),
        ],
        out_specs=pl.BlockSpec((TR, N), lambda j: (j, 0)),
        out_shape=jax.ShapeDtypeStruct((N, N), jnp.float32),
    )


def kernel(x, pseudo, edge_index, W1, Wr1, b1, W2, Wr2, b2, W3, Wr3, b3,
           W4, Wr4, b4, W5, Wr5, b5, W6, Wr6, b6, lin1_W, lin1_b, lin2_W, lin2_b):
    s_sorted, d_sorted, w_sorted, block_k, deg = _prepare_pairs_pallas(
        pseudo, edge_index)
    w_blk = w_sorted.reshape(1, P_CAP)

    xt = jnp.zeros((1, NPAD), jnp.float32).at[:, :N].set(x.T)
    convs = [(W1, Wr1, b1), (W2, Wr2, b2), (W3, Wr3, b3),
             (W4, Wr4, b4), (W5, Wr5, b5), (W6, Wr6, b6)]
    for (W, Wr, b) in convs:
        din, dout = W.shape[1], W.shape[2]
        xgt = _sc_gather(din)(xt, s_sorted)
        mt = _tc_matmul(din, dout)(block_k, xgt, W, W, W, W, w_blk)
        numt = _sc_scatter(dout)(mt, d_sorted)
        xt = _tc_epilogue(din, dout)(numt, deg, xt, Wr, b.reshape(dout, 1))

    h1t = _tc_lin1()(xt, lin1_W, lin1_b.reshape(256, 1))
    out = _tc_head()(h1t, lin2_W, lin2_b.reshape(1, N))
    return out


# speedup vs baseline: 8.6131x; 1.2009x over previous
import functools

import jax
import jax.numpy as jnp
from jax import lax
from jax.experimental import pallas as pl
from jax.experimental.pallas import tpu as pltpu
from jax.experimental.pallas import tpu_sc as plsc

N = 6890
E = 41340
KS = 5
KC = 125
NPAD = 6912
B = 256
NP = 8 * E
CHUNK = 12288
P_CAP = 368640
NBLK = P_CAP // B
NC, NS, LANES = 2, 16, 16
NW = NC * NS


E_PAD = 41472
EW = E_PAD // NW
NGRP = EW // LANES
NP_PAD = 8 * E_PAD
PT = NP_PAD // NS
CH2 = 6912
HALF = P_CAP // 2
HALF16 = HALF // NS
NBLK_PAD = 1536
OFFC = [(c & 1) * 25 + ((c >> 1) & 1) * 5 + ((c >> 2) & 1) for c in range(8)]

_SC_MESH = dict(core_axis_name="c", subcore_axis_name="s",
                num_cores=NC, num_subcores=NS)
_SC_PARAMS = dict(compiler_params=pltpu.CompilerParams(needs_layout_passes=False))


def _corner_w(c, f0, f1, f2):
    t0 = f0 if (c & 1) else 1.0 - f0
    t1 = f1 if ((c >> 1) & 1) else 1.0 - f1
    t2 = f2 if ((c >> 2) & 1) else 1.0 - f2
    return t0 * t1 * t2


def _spline_loop_a(ps_v, f0_v, f1_v, f2_v, lb_v, iota):
    @plsc.parallel_loop(0, EW, LANES, unroll=4)
    def la(t):
        idx3 = (t + iota) * 3
        u0 = plsc.load_gather(ps_v, [idx3]) * (KS - 1.0)
        u1 = plsc.load_gather(ps_v, [idx3 + 1]) * (KS - 1.0)
        u2 = plsc.load_gather(ps_v, [idx3 + 2]) * (KS - 1.0)
        l0 = jnp.minimum(u0.astype(jnp.int32), KS - 2)
        l1 = jnp.minimum(u1.astype(jnp.int32), KS - 2)
        l2 = jnp.minimum(u2.astype(jnp.int32), KS - 2)
        f0_v[pl.ds(t, LANES)] = u0 - l0.astype(jnp.float32)
        f1_v[pl.ds(t, LANES)] = u1 - l1.astype(jnp.float32)
        f2_v[pl.ds(t, LANES)] = u2 - l2.astype(jnp.float32)
        lb_v[pl.ds(t, LANES)] = l0 * 25 + l1 * 5 + l2


def _sc_prep_hist():
    mesh = plsc.VectorSubcoreMesh(**_SC_MESH)

    @functools.partial(
        pl.kernel, mesh=mesh, **_SC_PARAMS,
        out_type=(jax.ShapeDtypeStruct((NW, 128), jnp.int32),
                  jax.ShapeDtypeStruct((NW, NPAD), jnp.float32)),
        scratch_types=[
            pltpu.VMEM((EW * 3,), jnp.float32),
            pltpu.VMEM((EW,), jnp.int32),
            pltpu.VMEM((EW,), jnp.float32),
            pltpu.VMEM((EW,), jnp.float32),
            pltpu.VMEM((EW,), jnp.float32),
            pltpu.VMEM((EW,), jnp.int32),
            pltpu.VMEM((128,), jnp.int32),
            pltpu.VMEM((NPAD,), jnp.float32),
        ],
    )
    def k(ps_hbm, dst_hbm, hist_hbm, degp_hbm,
          ps_v, dst_v, f0_v, f1_v, f2_v, lb_v, hist_v, degp_v):
        wid = lax.axis_index("s") * NC + lax.axis_index("c")
        e0 = wid * EW
        iota = lax.broadcasted_iota(jnp.int32, (LANES,), 0)
        pltpu.sync_copy(ps_hbm.at[pl.ds(e0 * 3, EW * 3)], ps_v)
        pltpu.sync_copy(dst_hbm.at[pl.ds(e0, EW)], dst_v)

        @plsc.parallel_loop(0, 128, LANES)
        def z0(t):
            hist_v[pl.ds(t, LANES)] = jnp.zeros((LANES,), jnp.int32)

        @plsc.parallel_loop(0, NPAD, LANES, unroll=8)
        def z1(t):
            degp_v[pl.ds(t, LANES)] = jnp.zeros((LANES,), jnp.float32)

        _spline_loop_a(ps_v, f0_v, f1_v, f2_v, lb_v, iota)
        ones_f = jnp.ones((LANES,), jnp.float32)

        @plsc.parallel_loop(0, EW, LANES)
        def lb(t):
            kb = lb_v[pl.ds(t, LANES)]
            ev = (t + iota + e0) < E
            dd = dst_v[pl.ds(t, LANES)]
            plsc.addupdate_scatter(degp_v, [dd], ones_f, mask=ev)
            for c in range(8):
                kc = kb + OFFC[c]
                cnts, last = plsc.scan_count(kc)
                plsc.addupdate_scatter(hist_v, [kc], cnts, mask=last)

        pltpu.sync_copy(hist_v, hist_hbm.at[wid])
        pltpu.sync_copy(degp_v, degp_hbm.at[wid])
    return k


def _tc_prep_mid():
    def body(hist_ref, degp_ref, base_ref, blockk_ref, deg_ref):
        h = hist_ref[...].astype(jnp.float32)
        tot = jnp.sum(h, axis=0, keepdims=True)
        padded = jnp.floor((tot + (B - 1)) * (1.0 / B)).astype(jnp.float32)
        padded = padded * B
        r128 = lax.broadcasted_iota(jnp.int32, (128, 128), 0)
        c128 = lax.broadcasted_iota(jnp.int32, (128, 128), 1)
        lt128 = (r128 < c128).astype(jnp.float32)
        offs = lax.dot_general(padded, lt128, (((1,), (0,)), ((), ())),
                               preferred_element_type=jnp.float32)
        r32 = lax.broadcasted_iota(jnp.int32, (NW, NW), 0)
        c32 = lax.broadcasted_iota(jnp.int32, (NW, NW), 1)
        lt32 = (r32 > c32).astype(jnp.float32)
        prev = lax.dot_general(lt32, h, (((1,), (0,)), ((), ())),
                               preferred_element_type=jnp.float32)
        base_ref[...] = (offs + prev).astype(jnp.int32)

        jb = (lax.broadcasted_iota(jnp.int32, (NBLK_PAD, 128), 0) * B)
        hit = (offs.astype(jnp.int32) <= jb).astype(jnp.float32)
        nk = jnp.sum(hit, axis=1, keepdims=True).astype(jnp.int32) - 1
        blockk_ref[...] = jnp.clip(nk, 0, KC - 1)

        deg_ref[...] = jnp.maximum(jnp.sum(degp_ref[...], axis=0,
                                           keepdims=True), 1.0)

    return pl.pallas_call(
        body,
        out_shape=(jax.ShapeDtypeStruct((NW, 128), jnp.int32),
                   jax.ShapeDtypeStruct((NBLK_PAD, 1), jnp.int32),
                   jax.ShapeDtypeStruct((1, NPAD), jnp.float32)),
    )


def _sc_prep_pos():
    mesh = plsc.VectorSubcoreMesh(**_SC_MESH)

    @functools.partial(
        pl.kernel, mesh=mesh, **_SC_PARAMS,
        out_type=(jax.ShapeDtypeStruct((NP_PAD,), jnp.int32),
                  jax.ShapeDtypeStruct((NP_PAD,), jnp.float32)),
        scratch_types=[
            pltpu.VMEM((EW * 3,), jnp.float32),
            pltpu.VMEM((EW,), jnp.float32),
            pltpu.VMEM((EW,), jnp.float32),
            pltpu.VMEM((EW,), jnp.float32),
            pltpu.VMEM((EW,), jnp.int32),
            pltpu.VMEM((128,), jnp.int32),
            pltpu.VMEM((8 * EW,), jnp.int32),
            pltpu.VMEM((8 * EW,), jnp.float32),
        ],
    )
    def k(ps_hbm, base_hbm, pos8_hbm, w8_hbm,
          ps_v, f0_v, f1_v, f2_v, lb_v, cnt_v, pos_v, ww_v):
        wid = lax.axis_index("s") * NC + lax.axis_index("c")
        e0 = wid * EW
        iota = lax.broadcasted_iota(jnp.int32, (LANES,), 0)
        pltpu.sync_copy(ps_hbm.at[pl.ds(e0 * 3, EW * 3)], ps_v)
        pltpu.sync_copy(base_hbm.at[pl.ds(wid * 128, 128)], cnt_v)
        _spline_loop_a(ps_v, f0_v, f1_v, f2_v, lb_v, iota)

        def lb(g, carry):
            t = g * LANES
            kb = lb_v[pl.ds(t, LANES)]
            f0 = f0_v[pl.ds(t, LANES)]
            f1 = f1_v[pl.ds(t, LANES)]
            f2 = f2_v[pl.ds(t, LANES)]
            evf = jnp.where((t + iota + e0) < E, 1.0, 0.0)
            for c in range(8):
                kc = kb + OFFC[c]
                wc = _corner_w(c, f0, f1, f2) * evf
                cnts, last = plsc.scan_count(kc)
                basev = plsc.load_gather(cnt_v, [kc])
                pos_v[pl.ds(c * EW + t, LANES)] = basev + cnts - 1
                ww_v[pl.ds(c * EW + t, LANES)] = wc
                plsc.addupdate_scatter(cnt_v, [kc], cnts, mask=last)
            return carry
        lax.fori_loop(0, NGRP, lb, 0)

        for c in range(8):
            pltpu.sync_copy(pos_v.at[pl.ds(c * EW, EW)],
                            pos8_hbm.at[pl.ds(c * E_PAD + e0, EW)])
            pltpu.sync_copy(ww_v.at[pl.ds(c * EW, EW)],
                            w8_hbm.at[pl.ds(c * E_PAD + e0, EW)])
    return k


def _sc_prep_scatter():
    mesh = plsc.VectorSubcoreMesh(**_SC_MESH)
    n_ch = PT // CH2
    rows = CH2 // 128

    @functools.partial(
        pl.kernel, mesh=mesh, **_SC_PARAMS,
        out_type=(jax.ShapeDtypeStruct((P_CAP,), jnp.int32),
                  jax.ShapeDtypeStruct((P_CAP,), jnp.int32),
                  jax.ShapeDtypeStruct((P_CAP,), jnp.float32)),
        scratch_types=[
            pltpu.VMEM((CH2,), jnp.int32),
            pltpu.VMEM((CH2,), jnp.int32),
            pltpu.VMEM((CH2,), jnp.int32),
            pltpu.VMEM((CH2,), jnp.float32),
            pltpu.VMEM((CH2,), jnp.int32),
            pltpu.VMEM_SHARED((HALF + 8,), jnp.int32),
            pltpu.VMEM_SHARED((HALF + 8,), jnp.int32),
            pltpu.VMEM_SHARED((HALF + 8,), jnp.float32),
        ],
    )
    def k(pos8_hbm, w8_hbm, src_hbm, dst_hbm, sinit_hbm, winit_hbm,
          sout_hbm, dout_hbm, wout_hbm,
          pos_v, sv_v, dv_v, wv_v, lidx_v, s_sp, d_sp, w_sp):
        ci = lax.axis_index("c")
        si = lax.axis_index("s")
        gbase = ci * HALF + si * HALF16
        lbase = si * HALF16
        pltpu.sync_copy(sinit_hbm.at[pl.ds(gbase, HALF16)],
                        s_sp.at[pl.ds(lbase, HALF16)])
        pltpu.sync_copy(sinit_hbm.at[pl.ds(gbase, HALF16)],
                        d_sp.at[pl.ds(lbase, HALF16)])
        pltpu.sync_copy(winit_hbm.at[pl.ds(gbase, HALF16)],
                        w_sp.at[pl.ds(lbase, HALF16)])
        plsc.subcore_barrier()

        iota = lax.broadcasted_iota(jnp.int32, (LANES,), 0)
        half_lo = ci * HALF
        for j in range(n_ch):
            pc = si * PT + j * CH2
            ec = pc - (pc // E_PAD) * E_PAD
            pltpu.sync_copy(pos8_hbm.at[pl.ds(pc, CH2)], pos_v)
            pltpu.sync_copy(w8_hbm.at[pl.ds(pc, CH2)], wv_v)
            pltpu.sync_copy(src_hbm.at[pl.ds(ec, CH2)], sv_v)
            pltpu.sync_copy(dst_hbm.at[pl.ds(ec, CH2)], dv_v)

            @plsc.parallel_loop(0, CH2, LANES, unroll=4)
            def pb(t):
                p = pos_v[pl.ds(t, LANES)]
                inh = (p >= half_lo) & (p < half_lo + HALF)
                lidx_v[pl.ds(t, LANES)] = jnp.where(inh, p - half_lo, HALF)

            pltpu.sync_copy(sv_v, s_sp.at[lidx_v])
            pltpu.sync_copy(dv_v, d_sp.at[lidx_v])
            pltpu.sync_copy(wv_v, w_sp.at[lidx_v])
        plsc.subcore_barrier()
        pltpu.sync_copy(s_sp.at[pl.ds(lbase, HALF16)],
                        sout_hbm.at[pl.ds(gbase, HALF16)])
        pltpu.sync_copy(d_sp.at[pl.ds(lbase, HALF16)],
                        dout_hbm.at[pl.ds(gbase, HALF16)])
        pltpu.sync_copy(w_sp.at[pl.ds(lbase, HALF16)],
                        wout_hbm.at[pl.ds(gbase, HALF16)])
    return k


def _prepare_pairs_pallas(pseudo, edge_index):
    ps_flat = jnp.zeros((E_PAD * 3,), jnp.float32).at[:E * 3].set(
        pseudo.reshape(-1))
    src_pad = jnp.zeros((E_PAD,), jnp.int32).at[:E].set(edge_index[0])
    dst_pad = jnp.zeros((E_PAD,), jnp.int32).at[:E].set(edge_index[1])

    hist, degp = _sc_prep_hist()(ps_flat, dst_pad)
    base, blockk, deg = _tc_prep_mid()(hist, degp)
    pos8, w8 = _sc_prep_pos()(ps_flat, base.reshape(-1))
    sinit = jnp.arange(P_CAP, dtype=jnp.int32) % N
    winit = jnp.zeros((P_CAP,), jnp.float32)
    s_sorted, d_sorted, w_sorted = _sc_prep_scatter()(
        pos8, w8, src_pad, dst_pad, sinit, winit)
    block_k = blockk.reshape(-1)[:NBLK]
    return s_sorted, d_sorted, w_sorted, block_k, deg


def _sc_gather(din):
    mesh = plsc.VectorSubcoreMesh(core_axis_name="c", subcore_axis_name="s",
                                  num_cores=NC, num_subcores=NS)
    n_chunks = P_CAP // CHUNK

    if din >= NW:
        R = din // NW

        @functools.partial(
            pl.kernel, mesh=mesh,
            compiler_params=pltpu.CompilerParams(needs_layout_passes=False),
            out_type=jax.ShapeDtypeStruct((din, P_CAP), jnp.float32),
            scratch_types=[
                pltpu.VMEM((R * NPAD,), jnp.float32),
                pltpu.VMEM((2, CHUNK), jnp.int32),
                pltpu.VMEM((2, R, CHUNK), jnp.float32),
                pltpu.SemaphoreType.DMA((2,)),
                pltpu.SemaphoreType.DMA((2,)),
            ],
        )
        def k(xt_hbm, s_hbm, xgt_hbm, xrow_v, sidx_v, out_v, sem_in, sem_out):
            wid = lax.axis_index("s") * NC + lax.axis_index("c")
            row0 = wid * R
            for r in range(R):
                pltpu.sync_copy(xt_hbm.at[row0 + r],
                                xrow_v.at[pl.ds(r * NPAD, NPAD)])

            pltpu.async_copy(s_hbm.at[pl.ds(0, CHUNK)], sidx_v.at[0],
                             sem_in.at[0])

            def chunk_body(gg, _):
                for b in range(2):
                    g = gg * 2 + b
                    base = g * CHUNK
                    pltpu.make_async_copy(s_hbm.at[pl.ds(0, CHUNK)],
                                          sidx_v.at[b], sem_in.at[b]).wait()

                    @pl.when(g + 1 < n_chunks)
                    def _():
                        pltpu.async_copy(
                            s_hbm.at[pl.ds((g + 1) * CHUNK, CHUNK)],
                            sidx_v.at[1 - b], sem_in.at[1 - b])

                    @pl.when(g >= 2)
                    def _():
                        pltpu.make_async_copy(
                            out_v.at[b],
                            xgt_hbm.at[pl.ds(row0, R), pl.ds(0, CHUNK)],
                            sem_out.at[b]).wait()

                    @plsc.parallel_loop(0, CHUNK, LANES, unroll=8)
                    def body(t):
                        idx = sidx_v[b, pl.ds(t, LANES)]
                        for r in range(R):
                            vals = plsc.load_gather(
                                xrow_v, [idx + jnp.int32(r * NPAD)])
                            out_v[b, r, pl.ds(t, LANES)] = vals
                    pltpu.async_copy(
                        out_v.at[b],
                        xgt_hbm.at[pl.ds(row0, R), pl.ds(base, CHUNK)],
                        sem_out.at[b])
                return 0
            lax.fori_loop(0, n_chunks // 2, chunk_body, 0)
            for b in range(2):
                pltpu.make_async_copy(
                    out_v.at[b], xgt_hbm.at[pl.ds(row0, R), pl.ds(0, CHUNK)],
                    sem_out.at[b]).wait()
        return k

    per_w = P_CAP // NW

    @functools.partial(
        pl.kernel, mesh=mesh,
        compiler_params=pltpu.CompilerParams(needs_layout_passes=False),
        out_type=jax.ShapeDtypeStruct((din, P_CAP), jnp.float32),
        scratch_types=[
            pltpu.VMEM((NPAD,), jnp.float32),
            pltpu.VMEM((per_w,), jnp.int32),
            pltpu.VMEM((per_w,), jnp.float32),
        ],
    )
    def k1(xt_hbm, s_hbm, xgt_hbm, xrow_v, sidx_v, out_v):
        wid = lax.axis_index("s") * NC + lax.axis_index("c")
        base = wid * per_w
        pltpu.sync_copy(xt_hbm.at[0], xrow_v)
        pltpu.sync_copy(s_hbm.at[pl.ds(base, per_w)], sidx_v)

        @plsc.parallel_loop(0, per_w, LANES, unroll=8)
        def body(t):
            idx = sidx_v[pl.ds(t, LANES)]
            vals = plsc.load_gather(xrow_v, [idx])
            out_v[pl.ds(t, LANES)] = vals
        pltpu.sync_copy(out_v, xgt_hbm.at[0, pl.ds(base, per_w)])
    return k1


def _sc_scatter(dout):
    mesh = plsc.VectorSubcoreMesh(core_axis_name="c", subcore_axis_name="s",
                                  num_cores=NC, num_subcores=NS)
    n_chunks = P_CAP // CHUNK
    R = dout // NW
    assert R >= 1

    @functools.partial(
        pl.kernel, mesh=mesh,
        compiler_params=pltpu.CompilerParams(needs_layout_passes=False),
        out_type=jax.ShapeDtypeStruct((dout, NPAD), jnp.float32),
        scratch_types=[
            pltpu.VMEM((R * NPAD,), jnp.float32),
            pltpu.VMEM((2, CHUNK), jnp.int32),
            pltpu.VMEM((2, R, CHUNK), jnp.float32),
            pltpu.SemaphoreType.DMA((2,)),
            pltpu.SemaphoreType.DMA((2,)),
        ],
    )
    def k(mt_hbm, d_hbm, numt_hbm, acc_v, didx_v, m_v, sem_d, sem_m):
        wid = lax.axis_index("s") * NC + lax.axis_index("c")
        row0 = wid * R

        @plsc.parallel_loop(0, R * NPAD, LANES, unroll=8)
        def zero_body(t):
            acc_v[pl.ds(t, LANES)] = jnp.zeros((LANES,), jnp.float32)

        pltpu.async_copy(d_hbm.at[pl.ds(0, CHUNK)], didx_v.at[0], sem_d.at[0])
        pltpu.async_copy(mt_hbm.at[pl.ds(row0, R), pl.ds(0, CHUNK)],
                         m_v.at[0], sem_m.at[0])

        def chunk_body(gg, _):
            for b in range(2):
                g = gg * 2 + b
                pltpu.make_async_copy(d_hbm.at[pl.ds(0, CHUNK)],
                                      didx_v.at[b], sem_d.at[b]).wait()
                pltpu.make_async_copy(
                    mt_hbm.at[pl.ds(row0, R), pl.ds(0, CHUNK)],
                    m_v.at[b], sem_m.at[b]).wait()

                @pl.when(g + 1 < n_chunks)
                def _():
                    nbase = (g + 1) * CHUNK
                    pltpu.async_copy(d_hbm.at[pl.ds(nbase, CHUNK)],
                                     didx_v.at[1 - b], sem_d.at[1 - b])
                    pltpu.async_copy(
                        mt_hbm.at[pl.ds(row0, R), pl.ds(nbase, CHUNK)],
                        m_v.at[1 - b], sem_m.at[1 - b])

                @plsc.parallel_loop(0, CHUNK, LANES, unroll=8)
                def body(t):
                    idx = didx_v[b, pl.ds(t, LANES)]
                    for r in range(R):
                        vals = m_v[b, r, pl.ds(t, LANES)]
                        plsc.addupdate_scatter(acc_v, [idx + jnp.int32(r * NPAD)], vals)
            return 0
        lax.fori_loop(0, n_chunks // 2, chunk_body, 0)
        for r in range(R):
            pltpu.sync_copy(acc_v.at[pl.ds(r * NPAD, NPAD)], numt_hbm.at[row0 + r])
    return k


def _tc_matmul(din, dout):
    UB = 8

    def body(block_k_ref, xg_ref, *rest):
        w_refs = rest[:UB]
        wgt_ref, out_ref = rest[UB], rest[UB + 1]
        for u in range(UB):
            wk = w_refs[u][...].astype(jnp.bfloat16)
            xg = xg_ref[:, u * B:(u + 1) * B].astype(jnp.bfloat16)
            m = lax.dot_general(wk, xg, (((0,), (0,)), ((), ())),
                                preferred_element_type=jnp.float32)
            out_ref[:, u * B:(u + 1) * B] = m * wgt_ref[:, u * B:(u + 1) * B]

    def mk_wspec(u):
        return pl.BlockSpec((None, din, dout),
                            lambda j, bk, u=u: (bk[j * UB + u], 0, 0))

    grid_spec = pltpu.PrefetchScalarGridSpec(
        num_scalar_prefetch=1,
        grid=(NBLK // UB,),
        in_specs=[
            pl.BlockSpec((din, UB * B), lambda j, bk: (0, j)),
        ] + [mk_wspec(u) for u in range(UB)] + [
            pl.BlockSpec((1, UB * B), lambda j, bk: (0, j)),
        ],
        out_specs=pl.BlockSpec((dout, UB * B), lambda j, bk: (0, j)),
    )
    return pl.pallas_call(
        body, grid_spec=grid_spec,
        out_shape=jax.ShapeDtypeStruct((dout, P_CAP), jnp.float32),
    )


TN = 768


def _tc_epilogue(din, dout):
    def body(num_ref, deg_ref, xt_ref, wr_ref, b_ref, out_ref):
        agg = num_ref[...] / deg_ref[...]
        root = lax.dot_general(wr_ref[...], xt_ref[...], (((0,), (0,)), ((), ())),
                               preferred_element_type=jnp.float32)
        h = agg + root + b_ref[...]
        out_ref[...] = jnp.where(h > 0, h, jnp.exp(h) - 1.0)

    return pl.pallas_call(
        body,
        grid=(NPAD // TN,),
        in_specs=[
            pl.BlockSpec((dout, TN), lambda j: (0, j)),
            pl.BlockSpec((1, TN), lambda j: (0, j)),
            pl.BlockSpec((din, TN), lambda j: (0, j)),
            pl.BlockSpec((din, dout), lambda j: (0, 0)),
            pl.BlockSpec((dout, 1), lambda j: (0, 0)),
        ],
        out_specs=pl.BlockSpec((dout, TN), lambda j: (0, j)),
        out_shape=jax.ShapeDtypeStruct((dout, NPAD), jnp.float32),
    )


def _tc_lin1():
    def body(xt_ref, w_ref, b_ref, out_ref):
        h = lax.dot_general(w_ref[...], xt_ref[...], (((0,), (0,)), ((), ())),
                            preferred_element_type=jnp.float32) + b_ref[...]
        out_ref[...] = jnp.where(h > 0, h, jnp.exp(h) - 1.0)

    return pl.pallas_call(
        body,
        grid=(NPAD // TN,),
        in_specs=[
            pl.BlockSpec((64, TN), lambda j: (0, j)),
            pl.BlockSpec((64, 256), lambda j: (0, 0)),
            pl.BlockSpec((256, 1), lambda j: (0, 0)),
        ],
        out_specs=pl.BlockSpec((256, TN), lambda j: (0, j)),
        out_shape=jax.ShapeDtypeStruct((256, NPAD), jnp.float32),
    )


TR = 256


def _tc_head():
    def body(h_ref, w_ref, b_ref, out_ref):
        logits = lax.dot_general(h_ref[...].astype(jnp.bfloat16),
                                 w_ref[...].astype(jnp.bfloat16),
                                 (((0,), (0,)), ((), ())),
                                 preferred_element_type=jnp.float32)
        logits = logits + b_ref[...]
        m = jnp.max(logits, axis=1, keepdims=True)
        lse = m + jnp.log(jnp.sum(jnp.exp(logits - m), axis=1, keepdims=True))
        out_ref[...] = logits - lse

    nblk = (N + TR - 1) // TR
    return pl.pallas_call(
        body,
        grid=(nblk,),
        in_specs=[
            pl.BlockSpec((256, TR), lambda j: (0, j)),
            pl.BlockSpec((256, N), lambda j: (0, 0)),
            pl.BlockSpec((1, N), lambda j: (0, 0)),
        ],
        out_specs=pl.BlockSpec((TR, N), lambda j: (j, 0)),
        out_shape=jax.ShapeDtypeStruct((N, N), jnp.float32),
    )


def kernel(x, pseudo, edge_index, W1, Wr1, b1, W2, Wr2, b2, W3, Wr3, b3,
           W4, Wr4, b4, W5, Wr5, b5, W6, Wr6, b6, lin1_W, lin1_b, lin2_W, lin2_b):
    s_sorted, d_sorted, w_sorted, block_k, deg = _prepare_pairs_pallas(
        pseudo, edge_index)
    w_blk = w_sorted.reshape(1, P_CAP)

    xt = jnp.zeros((1, NPAD), jnp.float32).at[:, :N].set(x.T)
    convs = [(W1, Wr1, b1), (W2, Wr2, b2), (W3, Wr3, b3),
             (W4, Wr4, b4), (W5, Wr5, b5), (W6, Wr6, b6)]
    for (W, Wr, b) in convs:
        din, dout = W.shape[1], W.shape[2]
        xgt = _sc_gather(din)(xt, s_sorted)
        mt = _tc_matmul(din, dout)(block_k, xgt, *((W,) * 8), w_blk)
        numt = _sc_scatter(dout)(mt, d_sorted)
        xt = _tc_epilogue(din, dout)(numt, deg, xt, Wr, b.reshape(dout, 1))

    h1t = _tc_lin1()(xt, lin1_W, lin1_b.reshape(256, 1))
    out = _tc_head()(h1t, lin2_W, lin2_b.reshape(1, N))
    return out
